# Initial kernel scaffold; baseline (speedup 1.0000x reference)
import jax
import jax.numpy as jnp
from jax.experimental import pallas as pl


def kernel(input, tr_mask, tcl_mask, sin_map, cos_map, radii_map, train_mask):
    raise NotImplementedError("write your pallas kernel here")


# speedup vs baseline: 16.9051x; 16.9051x over previous
import functools

import jax
import jax.numpy as jnp
from jax import lax
from jax.experimental import pallas as pl
from jax.experimental.pallas import tpu as pltpu
from jax.experimental.pallas import tpu_sc as plsc

N = 8 * 512 * 512
NB = 4096
NBF = float(NB)
NW = 32
_H, _W = 512, 512
_RB = 64
_GB, _GR = 8, _H // _RB
_WR = 128


def _softplus(d):
    return jnp.maximum(d, 0.0) + jnp.log1p(jnp.exp(-jnp.abs(d)))


def _smooth_l1(x, y):
    d = jnp.abs(x - y)
    return jnp.where(d < 1.0, 0.5 * d * d, d - 0.5)


def _pass1_body(x_ref, tr_ref, tcl_ref, trn_ref, sin_ref, cos_ref, rad_ref,
                ce_ref, sc_ref, bnd_ref):
    b = pl.program_id(0)
    r = pl.program_id(1)

    l0 = x_ref[0, 0]
    l1 = x_ref[0, 1]
    t0 = x_ref[0, 2]
    t1 = x_ref[0, 3]
    sp = x_ref[0, 4]
    cp = x_ref[0, 5]
    rp = x_ref[0, 6]
    tr = tr_ref[0]
    tcl = tcl_ref[0]
    trn = trn_ref[0]
    sinm = sin_ref[0]
    cosm = cos_ref[0]
    radm = rad_ref[0]

    s = l1 - l0
    ce = _softplus(jnp.where(tr == 1, -s, s))
    posm = (tr * trn) != 0
    negm = ((1 - tr) * trn) != 0
    zero = jnp.zeros_like(ce)
    n_pos = jnp.sum(jnp.where(posm, 1.0, 0.0))
    loss_pos = jnp.sum(jnp.where(posm, ce, zero))
    n_negav = jnp.sum(jnp.where(negm, 1.0, 0.0))
    sum_neg = jnp.sum(jnp.where(negm, ce, zero))
    max_neg = jnp.max(jnp.where(negm, ce, jnp.full_like(ce, -jnp.inf)))
    ce_ref[0] = jnp.where(negm, ce, jnp.full_like(ce, -1.0))

    st = t1 - t0
    ce_t = _softplus(jnp.where(tcl == 1, -st, st))
    ttmm = (trn * tr) != 0
    ttm_cnt = jnp.sum(jnp.where(ttmm, 1.0, 0.0))
    sum_tcl = jnp.sum(jnp.where(ttmm, ce_t, zero))

    sel = tcl != 0
    m_cnt = jnp.sum(jnp.where(sel, 1.0, 0.0))
    tct_cnt = jnp.sum(jnp.where((trn * tcl) != 0, 1.0, 0.0))
    scale = jnp.sqrt(1.0 / (sp * sp + cp * cp))
    s_rad = jnp.sum(jnp.where(sel, _smooth_l1(rp / radm, jnp.ones_like(rp)), zero))
    s_sin = jnp.sum(jnp.where(sel, _smooth_l1(sp * scale, sinm), zero))
    s_cos = jnp.sum(jnp.where(sel, _smooth_l1(cp * scale, cosm), zero))

    vals = [n_pos, loss_pos, n_negav, sum_neg, None, ttm_cnt, sum_tcl,
            m_cnt, tct_cnt, s_rad, s_sin, s_cos]
    first = (b == 0) & (r == 0)

    @pl.when(first)
    def _():
        for i, v in enumerate(vals):
            if v is not None:
                sc_ref[0, i] = v
        sc_ref[0, 4] = max_neg

    @pl.when(jnp.logical_not(first))
    def _():
        for i, v in enumerate(vals):
            if v is not None:
                sc_ref[0, i] = sc_ref[0, i] + v
        sc_ref[0, 4] = jnp.maximum(sc_ref[0, 4], max_neg)

    @pl.when((b == _GB - 1) & (r == _GR - 1))
    def _():
        scale2 = NBF / jnp.maximum(sc_ref[0, 4], 1e-6)
        rows = lax.broadcasted_iota(jnp.int32, (8, 128), 0)
        bnd_ref[...] = jnp.where(rows == 0, scale2, 0.0)


def _run_pass1(x, tr, tcl, trn, sinm, cosm, radm):
    m_spec = pl.BlockSpec((1, _RB, _W), lambda b, r: (b, r, 0))
    return pl.pallas_call(
        _pass1_body,
        grid=(_GB, _GR),
        in_specs=[
            pl.BlockSpec((1, 7, _RB, _W), lambda b, r: (b, 0, r, 0)),
            m_spec, m_spec, m_spec, m_spec, m_spec, m_spec,
        ],
        out_specs=[
            pl.BlockSpec((1, _RB, _W),
                         lambda b, r: (b * 4 + r // 2, r % 2, 0)),
            pl.BlockSpec((1, 16), lambda b, r: (0, 0),
                         memory_space=pltpu.SMEM),
            pl.BlockSpec((8, 128), lambda b, r: (0, 0)),
        ],
        out_shape=[
            jax.ShapeDtypeStruct((NW, _WR, _W), jnp.float32),
            jax.ShapeDtypeStruct((1, 16), jnp.float32),
            jax.ShapeDtypeStruct((8, 128), jnp.float32),
        ],
    )(x, tr, tcl, trn, sinm, cosm, radm)


def _zero_hist(cnt_v):
    zeros = jnp.zeros((16,), jnp.float32)

    def zbody(i, c):
        for u in range(8):
            cnt_v[pl.ds(i * 128 + u * 16, 16)] = zeros
        return c

    lax.fori_loop(0, NB // 128, zbody, 0)


def _sc_hist2_body(x_hbm, bnd_hbm, cnt_hbm, buf, cnt_v, bnd_v):
    wid = lax.axis_index("s") * 2 + lax.axis_index("c")
    pltpu.sync_copy(x_hbm.at[wid], buf)
    pltpu.sync_copy(bnd_hbm, bnd_v)
    scale2 = bnd_v[0, pl.ds(0, 16)]
    _zero_hist(cnt_v)
    ones = jnp.ones((16,), jnp.float32)
    hi_clip = jnp.full((16,), NBF - 1.0, jnp.float32)

    def row(i, c):
        for u in range(_W // 16):
            v = buf[i, pl.ds(u * 16, 16)]
            t = jnp.minimum(jnp.maximum(v * scale2, 0.0), hi_clip)
            idx = t.astype(jnp.int32)
            plsc.addupdate_scatter(cnt_v, [idx], ones, mask=v >= 0.0)
        return c

    lax.fori_loop(0, _WR, row, 0)
    pltpu.sync_copy(cnt_v, cnt_hbm.at[wid])


def _sc_hist3_body(x_hbm, bnd_hbm, cnt_hbm, buf, cnt_v, bnd_v):
    wid = lax.axis_index("s") * 2 + lax.axis_index("c")
    pltpu.sync_copy(x_hbm.at[wid], buf)
    pltpu.sync_copy(bnd_hbm, bnd_v)
    scale2 = bnd_v[0, pl.ds(0, 16)]
    bsel = bnd_v[1, pl.ds(0, 16)].astype(jnp.int32)
    lo = bnd_v[2, pl.ds(0, 16)]
    scale3 = bnd_v[3, pl.ds(0, 16)]
    _zero_hist(cnt_v)
    ones = jnp.ones((16,), jnp.float32)
    hi_clip = jnp.full((16,), NBF - 1.0, jnp.float32)

    def row(i, c):
        for u in range(_W // 16):
            v = buf[i, pl.ds(u * 16, 16)]
            t2 = jnp.minimum(jnp.maximum(v * scale2, 0.0), hi_clip)
            i2 = t2.astype(jnp.int32)
            t3 = jnp.minimum(jnp.maximum((v - lo) * scale3, 0.0), hi_clip)
            idx = t3.astype(jnp.int32)
            valid = (v >= 0.0) & (i2 == bsel)
            plsc.addupdate_scatter(cnt_v, [idx], ones, mask=valid)
        return c

    lax.fori_loop(0, _WR, row, 0)
    pltpu.sync_copy(cnt_v, cnt_hbm.at[wid])


@functools.cache
def _get_sc_hist(level):
    return pl.kernel(
        _sc_hist2_body if level == 2 else _sc_hist3_body,
        out_type=jax.ShapeDtypeStruct((NW, NB), jnp.float32),
        mesh=plsc.VectorSubcoreMesh(core_axis_name="c", subcore_axis_name="s"),
        compiler_params=pltpu.CompilerParams(needs_layout_passes=False),
        scratch_types=[
            pltpu.VMEM((_WR, _W), jnp.float32),
            pltpu.VMEM((NB,), jnp.float32),
            pltpu.VMEM((8, 128), jnp.float32),
        ],
    )


def _suffix_sum(x):
    s = x
    off = 1
    while off < NB:
        s = s + jnp.concatenate(
            [s[:, off:], jnp.zeros((1, off), s.dtype)], axis=1)
        off *= 2
    return s


def _select_body(sc_ref, cnt_ref, bnd_ref, car_ref):
    cnt = jnp.sum(cnt_ref[...], axis=0, keepdims=True)
    n_pos = sc_ref[0, 0]
    avail = sc_ref[0, 2]
    mx = sc_ref[0, 4]
    k = jnp.where(n_pos > 0, jnp.minimum(avail, 3.0 * n_pos), 100.0)
    cge = _suffix_sum(cnt)
    j = lax.broadcasted_iota(jnp.int32, (1, NB), 1).astype(jnp.float32)
    b = jnp.max(jnp.where(cge >= k, j, -1.0))
    cnt_gt = jnp.sum(jnp.where(j > b, cnt, 0.0))
    k_rem = k - cnt_gt
    scale2 = NBF / jnp.maximum(mx, 1e-6)
    lo_b = b / scale2
    scale3 = scale2 * NBF
    rows = lax.broadcasted_iota(jnp.int32, (8, 128), 0)
    bnd_ref[...] = jnp.where(
        rows == 0, scale2,
        jnp.where(rows == 1, b,
                  jnp.where(rows == 2, lo_b,
                            jnp.where(rows == 3, scale3, 0.0))))
    car_ref[0, 0] = k
    car_ref[0, 1] = k_rem
    car_ref[0, 2] = b
    car_ref[0, 3] = lo_b
    car_ref[0, 4] = scale2
    car_ref[0, 5] = scale3
    for i in range(6, 8):
        car_ref[0, i] = 0.0


def _run_select(sc, cnt2):
    return pl.pallas_call(
        _select_body,
        in_specs=[
            pl.BlockSpec(memory_space=pltpu.SMEM),
            pl.BlockSpec(memory_space=pltpu.MemorySpace.VMEM),
        ],
        out_specs=[
            pl.BlockSpec(memory_space=pltpu.MemorySpace.VMEM),
            pl.BlockSpec(memory_space=pltpu.SMEM),
        ],
        out_shape=[
            jax.ShapeDtypeStruct((8, 128), jnp.float32),
            jax.ShapeDtypeStruct((1, 8), jnp.float32),
        ],
    )(sc, cnt2)


def _select2_body(car_ref, cnt_ref, car2_ref):
    cnt = jnp.sum(cnt_ref[...], axis=0, keepdims=True)
    k_rem = car_ref[0, 1]
    cge = _suffix_sum(cnt)
    j = lax.broadcasted_iota(jnp.int32, (1, NB), 1).astype(jnp.float32)
    b3 = jnp.max(jnp.where(cge >= k_rem, j, -1.0))
    cnt_gt3 = jnp.sum(jnp.where(j > b3, cnt, 0.0))
    k3 = k_rem - cnt_gt3
    cnt_eq3 = jnp.sum(jnp.where(j == b3, cnt, 0.0))
    car2_ref[0, 0] = car_ref[0, 0]
    car2_ref[0, 1] = car_ref[0, 2]
    car2_ref[0, 2] = b3
    car2_ref[0, 3] = k3
    car2_ref[0, 4] = cnt_eq3
    car2_ref[0, 5] = car_ref[0, 4]
    car2_ref[0, 6] = car_ref[0, 3]
    car2_ref[0, 7] = car_ref[0, 5]


def _run_select2(carry, cnt3):
    return pl.pallas_call(
        _select2_body,
        in_specs=[
            pl.BlockSpec(memory_space=pltpu.SMEM),
            pl.BlockSpec(memory_space=pltpu.MemorySpace.VMEM),
        ],
        out_specs=pl.BlockSpec(memory_space=pltpu.SMEM),
        out_shape=jax.ShapeDtypeStruct((1, 8), jnp.float32),
    )(carry, cnt3)


def _final_body(sc_ref, car2_ref, ce_ref, out_ref):
    w = pl.program_id(0)
    scale2 = car2_ref[0, 5]
    lo_b = car2_ref[0, 6]
    scale3 = car2_ref[0, 7]
    b_i = car2_ref[0, 1].astype(jnp.int32)
    b3_i = car2_ref[0, 2].astype(jnp.int32)

    v = ce_ref[0]
    valid = v >= 0.0
    hi_clip = NBF - 1.0
    t2 = jnp.minimum(jnp.maximum(v * scale2, 0.0), hi_clip)
    i2 = t2.astype(jnp.int32)
    t3 = jnp.minimum(jnp.maximum((v - lo_b) * scale3, 0.0), hi_clip)
    i3 = t3.astype(jnp.int32)
    zero = jnp.zeros_like(v)
    eq2 = valid & (i2 == b_i)
    s_gt2 = jnp.sum(jnp.where(valid & (i2 > b_i), v, zero))
    s_gt3 = jnp.sum(jnp.where(eq2 & (i3 > b3_i), v, zero))
    s_eq3 = jnp.sum(jnp.where(eq2 & (i3 == b3_i), v, zero))

    @pl.when(w == 0)
    def _():
        out_ref[0, 5] = s_gt2
        out_ref[0, 6] = s_gt3
        out_ref[0, 7] = s_eq3

    @pl.when(w != 0)
    def _():
        out_ref[0, 5] = out_ref[0, 5] + s_gt2
        out_ref[0, 6] = out_ref[0, 6] + s_gt3
        out_ref[0, 7] = out_ref[0, 7] + s_eq3

    @pl.when(w == NW - 1)
    def _():
        k = car2_ref[0, 0]
        k3 = car2_ref[0, 3]
        cnt_eq3 = car2_ref[0, 4]
        mean3 = out_ref[0, 7] / jnp.maximum(cnt_eq3, 1.0)
        topk = out_ref[0, 5] + out_ref[0, 6] + k3 * mean3

        n_pos = sc_ref[0, 0]
        loss_pos = sc_ref[0, 1]
        avail = sc_ref[0, 2]
        sum_neg_tot = sc_ref[0, 3]
        loss_neg = jnp.where(k >= avail, sum_neg_tot,
                             jnp.where(k <= 0.0, 0.0, topk))
        loss_tr = (loss_pos + loss_neg) / (n_pos + k)

        ttm_cnt = sc_ref[0, 5]
        sum_tcl = sc_ref[0, 6]
        m_cnt = sc_ref[0, 7]
        tct_cnt = sc_ref[0, 8]
        loss_tcl = jnp.where(ttm_cnt > 0,
                             sum_tcl / jnp.maximum(ttm_cnt, 1.0), 0.0)
        gd = jnp.maximum(m_cnt, 1.0)
        out_ref[0, 0] = loss_tr
        out_ref[0, 1] = loss_tcl
        out_ref[0, 2] = jnp.where(tct_cnt > 0, sc_ref[0, 9] / gd, 0.0)
        out_ref[0, 3] = jnp.where(tct_cnt > 0, sc_ref[0, 10] / gd, 0.0)
        out_ref[0, 4] = jnp.where(tct_cnt > 0, sc_ref[0, 11] / gd, 0.0)


def _run_final(sc, carry2, ce_neg):
    return pl.pallas_call(
        _final_body,
        grid=(NW,),
        in_specs=[
            pl.BlockSpec((1, 16), lambda w: (0, 0), memory_space=pltpu.SMEM),
            pl.BlockSpec((1, 8), lambda w: (0, 0), memory_space=pltpu.SMEM),
            pl.BlockSpec((1, _WR, _W), lambda w: (w, 0, 0)),
        ],
        out_specs=pl.BlockSpec((1, 8), lambda w: (0, 0),
                               memory_space=pltpu.SMEM),
        out_shape=jax.ShapeDtypeStruct((1, 8), jnp.float32),
    )(sc, carry2, ce_neg)


def kernel(input, tr_mask, tcl_mask, sin_map, cos_map, radii_map, train_mask):
    ce_neg, sc, bnd2 = _run_pass1(input, tr_mask, tcl_mask, train_mask,
                                  sin_map, cos_map, radii_map)
    cnt2 = _get_sc_hist(2)(ce_neg, bnd2)
    bnd3, carry = _run_select(sc, cnt2)
    cnt3 = _get_sc_hist(3)(ce_neg, bnd3)
    carry2 = _run_select2(carry, cnt3)
    out = _run_final(sc, carry2, ce_neg)
    return (out[0, 0], out[0, 1], out[0, 2], out[0, 3], out[0, 4])


# speedup vs baseline: 16.9197x; 1.0009x over previous
import functools

import jax
import jax.numpy as jnp
from jax import lax
from jax.experimental import pallas as pl
from jax.experimental.pallas import tpu as pltpu
from jax.experimental.pallas import tpu_sc as plsc

N = 8 * 512 * 512
NB = 4096
NW = 32
_H, _W = 512, 512
_RB = 64
_GB, _GR = 8, _H // _RB
_WR = 128
_TRASH = 4095 << 12
_TBIN = 4080


def _softplus(d):
    return jnp.maximum(d, 0.0) + jnp.log1p(jnp.exp(-jnp.abs(d)))


def _smooth_l1(x, y):
    d = jnp.abs(x - y)
    return jnp.where(d < 1.0, 0.5 * d * d, d - 0.5)


def _pass1_body(x_ref, tr_ref, tcl_ref, trn_ref, sin_ref, cos_ref, rad_ref,
                ce_ref, key_ref, sc_ref):
    b = pl.program_id(0)
    r = pl.program_id(1)

    l0 = x_ref[0, 0]
    l1 = x_ref[0, 1]
    t0 = x_ref[0, 2]
    t1 = x_ref[0, 3]
    sp = x_ref[0, 4]
    cp = x_ref[0, 5]
    rp = x_ref[0, 6]
    tr = tr_ref[0]
    tcl = tcl_ref[0]
    trn = trn_ref[0]
    sinm = sin_ref[0]
    cosm = cos_ref[0]
    radm = rad_ref[0]

    s = l1 - l0
    ce = _softplus(jnp.where(tr == 1, -s, s))
    posm = (tr * trn) != 0
    negm = ((1 - tr) * trn) != 0
    zero = jnp.zeros_like(ce)
    n_pos = jnp.sum(jnp.where(posm, 1.0, 0.0))
    loss_pos = jnp.sum(jnp.where(posm, ce, zero))
    n_negav = jnp.sum(jnp.where(negm, 1.0, 0.0))
    sum_neg = jnp.sum(jnp.where(negm, ce, zero))
    ce_ref[0] = jnp.where(negm, ce, jnp.full_like(ce, -1.0))
    kbits = lax.shift_right_logical(lax.bitcast_convert_type(ce, jnp.int32), 7)
    key_ref[0] = jnp.where(negm, kbits, jnp.full_like(kbits, _TRASH))

    st = t1 - t0
    ce_t = _softplus(jnp.where(tcl == 1, -st, st))
    ttmm = (trn * tr) != 0
    ttm_cnt = jnp.sum(jnp.where(ttmm, 1.0, 0.0))
    sum_tcl = jnp.sum(jnp.where(ttmm, ce_t, zero))

    sel = tcl != 0
    m_cnt = jnp.sum(jnp.where(sel, 1.0, 0.0))
    tct_cnt = jnp.sum(jnp.where((trn * tcl) != 0, 1.0, 0.0))
    scale = jnp.sqrt(1.0 / (sp * sp + cp * cp))
    s_rad = jnp.sum(jnp.where(sel, _smooth_l1(rp / radm, jnp.ones_like(rp)), zero))
    s_sin = jnp.sum(jnp.where(sel, _smooth_l1(sp * scale, sinm), zero))
    s_cos = jnp.sum(jnp.where(sel, _smooth_l1(cp * scale, cosm), zero))

    vals = [n_pos, loss_pos, n_negav, sum_neg, ttm_cnt, sum_tcl,
            m_cnt, tct_cnt, s_rad, s_sin, s_cos]
    first = (b == 0) & (r == 0)

    @pl.when(first)
    def _():
        for i, v in enumerate(vals):
            sc_ref[0, i] = v

    @pl.when(jnp.logical_not(first))
    def _():
        for i, v in enumerate(vals):
            sc_ref[0, i] = sc_ref[0, i] + v


def _run_pass1(x, tr, tcl, trn, sinm, cosm, radm):
    m_spec = pl.BlockSpec((1, _RB, _W), lambda b, r: (b, r, 0))
    w_spec = pl.BlockSpec((1, _RB, _W), lambda b, r: (b * 4 + r // 2, r % 2, 0))
    return pl.pallas_call(
        _pass1_body,
        grid=(_GB, _GR),
        in_specs=[
            pl.BlockSpec((1, 7, _RB, _W), lambda b, r: (b, 0, r, 0)),
            m_spec, m_spec, m_spec, m_spec, m_spec, m_spec,
        ],
        out_specs=[
            w_spec,
            w_spec,
            pl.BlockSpec((1, 16), lambda b, r: (0, 0),
                         memory_space=pltpu.SMEM),
        ],
        out_shape=[
            jax.ShapeDtypeStruct((NW, _WR, _W), jnp.float32),
            jax.ShapeDtypeStruct((NW, _WR, _W), jnp.int32),
            jax.ShapeDtypeStruct((1, 16), jnp.float32),
        ],
    )(x, tr, tcl, trn, sinm, cosm, radm)


def _zero_hist(cnt_v):
    zeros = jnp.zeros((16,), jnp.float32)

    def zbody(i, c):
        for u in range(8):
            cnt_v[pl.ds(i * 128 + u * 16, 16)] = zeros
        return c

    lax.fori_loop(0, NB // 128, zbody, 0)


def _sc_hist2_body(key_hbm, cnt_hbm, buf, cnt_v):
    wid = lax.axis_index("s") * 2 + lax.axis_index("c")
    pltpu.sync_copy(key_hbm.at[wid], buf)
    _zero_hist(cnt_v)
    ones = jnp.ones((16,), jnp.float32)

    def row(i, c):
        for u in range(_W // 16):
            k = buf[i, pl.ds(u * 16, 16)]
            idx = lax.shift_right_logical(k, 12)
            plsc.addupdate_scatter(cnt_v, [idx], ones)
        return c

    lax.fori_loop(0, _WR, row, 0)
    pltpu.sync_copy(cnt_v, cnt_hbm.at[wid])


def _sc_hist3_body(key_hbm, bnd_hbm, cnt_hbm, buf, cnt_v, bnd_v):
    wid = lax.axis_index("s") * 2 + lax.axis_index("c")
    pltpu.sync_copy(key_hbm.at[wid], buf)
    pltpu.sync_copy(bnd_hbm, bnd_v)
    bsel = bnd_v[0, pl.ds(0, 16)].astype(jnp.int32)
    _zero_hist(cnt_v)
    ones = jnp.ones((16,), jnp.float32)
    low = jnp.full((16,), NB - 1, jnp.int32)

    def row(i, c):
        for u in range(_W // 16):
            k = buf[i, pl.ds(u * 16, 16)]
            coarse = lax.shift_right_logical(k, 12)
            idx = jnp.bitwise_and(k, low)
            plsc.addupdate_scatter(cnt_v, [idx], ones, mask=coarse == bsel)
        return c

    lax.fori_loop(0, _WR, row, 0)
    pltpu.sync_copy(cnt_v, cnt_hbm.at[wid])


@functools.cache
def _get_sc_hist(level):
    if level == 2:
        body = _sc_hist2_body
        scratch = [
            pltpu.VMEM((_WR, _W), jnp.int32),
            pltpu.VMEM((NB,), jnp.float32),
        ]
    else:
        body = _sc_hist3_body
        scratch = [
            pltpu.VMEM((_WR, _W), jnp.int32),
            pltpu.VMEM((NB,), jnp.float32),
            pltpu.VMEM((8, 128), jnp.float32),
        ]
    return pl.kernel(
        body,
        out_type=jax.ShapeDtypeStruct((NW, NB), jnp.float32),
        mesh=plsc.VectorSubcoreMesh(core_axis_name="c", subcore_axis_name="s"),
        compiler_params=pltpu.CompilerParams(needs_layout_passes=False),
        scratch_types=scratch,
    )


def _suffix_sum(x):
    s = x
    off = 1
    while off < NB:
        s = s + jnp.concatenate(
            [s[:, off:], jnp.zeros((1, off), s.dtype)], axis=1)
        off *= 2
    return s


def _select_body(sc_ref, cnt_ref, bnd_ref, car_ref):
    cnt = jnp.sum(cnt_ref[...], axis=0, keepdims=True)
    j = lax.broadcasted_iota(jnp.int32, (1, NB), 1).astype(jnp.float32)
    cnt = jnp.where(j < float(_TBIN), cnt, 0.0)
    n_pos = sc_ref[0, 0]
    avail = sc_ref[0, 2]
    k = jnp.where(n_pos > 0, jnp.minimum(avail, 3.0 * n_pos), 100.0)
    cge = _suffix_sum(cnt)
    b = jnp.max(jnp.where(cge >= k, j, -1.0))
    cnt_gt = jnp.sum(jnp.where(j > b, cnt, 0.0))
    k_rem = k - cnt_gt
    rows = lax.broadcasted_iota(jnp.int32, (8, 128), 0)
    bnd_ref[...] = jnp.where(rows == 0, b, 0.0)
    car_ref[0, 0] = k
    car_ref[0, 1] = k_rem
    car_ref[0, 2] = b
    for i in range(3, 8):
        car_ref[0, i] = 0.0


def _run_select(sc, cnt2):
    return pl.pallas_call(
        _select_body,
        in_specs=[
            pl.BlockSpec(memory_space=pltpu.SMEM),
            pl.BlockSpec(memory_space=pltpu.MemorySpace.VMEM),
        ],
        out_specs=[
            pl.BlockSpec(memory_space=pltpu.MemorySpace.VMEM),
            pl.BlockSpec(memory_space=pltpu.SMEM),
        ],
        out_shape=[
            jax.ShapeDtypeStruct((8, 128), jnp.float32),
            jax.ShapeDtypeStruct((1, 8), jnp.float32),
        ],
    )(sc, cnt2)


def _select2_body(car_ref, cnt_ref, car2_ref):
    cnt = jnp.sum(cnt_ref[...], axis=0, keepdims=True)
    k_rem = car_ref[0, 1]
    cge = _suffix_sum(cnt)
    j = lax.broadcasted_iota(jnp.int32, (1, NB), 1).astype(jnp.float32)
    b3 = jnp.max(jnp.where(cge >= k_rem, j, -1.0))
    cnt_gt3 = jnp.sum(jnp.where(j > b3, cnt, 0.0))
    k3 = k_rem - cnt_gt3
    cnt_eq3 = jnp.sum(jnp.where(j == b3, cnt, 0.0))
    car2_ref[0, 0] = car_ref[0, 0]
    car2_ref[0, 1] = car_ref[0, 2]
    car2_ref[0, 2] = b3
    car2_ref[0, 3] = k3
    car2_ref[0, 4] = cnt_eq3
    for i in range(5, 8):
        car2_ref[0, i] = 0.0


def _run_select2(carry, cnt3):
    return pl.pallas_call(
        _select2_body,
        in_specs=[
            pl.BlockSpec(memory_space=pltpu.SMEM),
            pl.BlockSpec(memory_space=pltpu.MemorySpace.VMEM),
        ],
        out_specs=pl.BlockSpec(memory_space=pltpu.SMEM),
        out_shape=jax.ShapeDtypeStruct((1, 8), jnp.float32),
    )(carry, cnt3)


def _final_body(sc_ref, car2_ref, ce_ref, out_ref):
    w = pl.program_id(0)
    b_i = car2_ref[0, 1].astype(jnp.int32)
    b3_i = car2_ref[0, 2].astype(jnp.int32)
    kthr = b_i * NB + b3_i

    v = ce_ref[0]
    valid = v >= 0.0
    key = lax.shift_right_logical(lax.bitcast_convert_type(v, jnp.int32), 7)
    zero = jnp.zeros_like(v)
    s_gt = jnp.sum(jnp.where(valid & (key > kthr), v, zero))
    s_eq = jnp.sum(jnp.where(valid & (key == kthr), v, zero))

    @pl.when(w == 0)
    def _():
        out_ref[0, 5] = s_gt
        out_ref[0, 6] = s_eq

    @pl.when(w != 0)
    def _():
        out_ref[0, 5] = out_ref[0, 5] + s_gt
        out_ref[0, 6] = out_ref[0, 6] + s_eq

    @pl.when(w == NW - 1)
    def _():
        k = car2_ref[0, 0]
        k3 = car2_ref[0, 3]
        cnt_eq3 = car2_ref[0, 4]
        mean3 = out_ref[0, 6] / jnp.maximum(cnt_eq3, 1.0)
        topk = out_ref[0, 5] + k3 * mean3

        n_pos = sc_ref[0, 0]
        loss_pos = sc_ref[0, 1]
        avail = sc_ref[0, 2]
        sum_neg_tot = sc_ref[0, 3]
        loss_neg = jnp.where(k >= avail, sum_neg_tot,
                             jnp.where(k <= 0.0, 0.0, topk))
        loss_tr = (loss_pos + loss_neg) / (n_pos + k)

        ttm_cnt = sc_ref[0, 4]
        sum_tcl = sc_ref[0, 5]
        m_cnt = sc_ref[0, 6]
        tct_cnt = sc_ref[0, 7]
        loss_tcl = jnp.where(ttm_cnt > 0,
                             sum_tcl / jnp.maximum(ttm_cnt, 1.0), 0.0)
        gd = jnp.maximum(m_cnt, 1.0)
        out_ref[0, 0] = loss_tr
        out_ref[0, 1] = loss_tcl
        out_ref[0, 2] = jnp.where(tct_cnt > 0, sc_ref[0, 8] / gd, 0.0)
        out_ref[0, 3] = jnp.where(tct_cnt > 0, sc_ref[0, 9] / gd, 0.0)
        out_ref[0, 4] = jnp.where(tct_cnt > 0, sc_ref[0, 10] / gd, 0.0)


def _run_final(sc, carry2, ce_neg):
    return pl.pallas_call(
        _final_body,
        grid=(NW,),
        in_specs=[
            pl.BlockSpec((1, 16), lambda w: (0, 0), memory_space=pltpu.SMEM),
            pl.BlockSpec((1, 8), lambda w: (0, 0), memory_space=pltpu.SMEM),
            pl.BlockSpec((1, _WR, _W), lambda w: (w, 0, 0)),
        ],
        out_specs=pl.BlockSpec((1, 8), lambda w: (0, 0),
                               memory_space=pltpu.SMEM),
        out_shape=jax.ShapeDtypeStruct((1, 8), jnp.float32),
    )(sc, carry2, ce_neg)


def kernel(input, tr_mask, tcl_mask, sin_map, cos_map, radii_map, train_mask):
    ce_neg, key, sc = _run_pass1(input, tr_mask, tcl_mask, train_mask,
                                 sin_map, cos_map, radii_map)
    cnt2 = _get_sc_hist(2)(key)
    bnd3, carry = _run_select(sc, cnt2)
    cnt3 = _get_sc_hist(3)(key, bnd3)
    carry2 = _run_select2(carry, cnt3)
    out = _run_final(sc, carry2, ce_neg)
    return (out[0, 0], out[0, 1], out[0, 2], out[0, 3], out[0, 4])


# speedup vs baseline: 20.6002x; 1.2175x over previous
import functools

import jax
import jax.numpy as jnp
from jax import lax
from jax.experimental import pallas as pl
from jax.experimental.pallas import tpu as pltpu
from jax.experimental.pallas import tpu_sc as plsc

N = 8 * 512 * 512
NB = 4096
NW = 32
_H, _W = 512, 512
_RB = 64
_GB, _GR = 8, _H // _RB
_WR = 128
_TRASH = 4095 << 12
_TBIN = 4080


def _softplus(d):
    return jnp.maximum(d, 0.0) + jnp.log1p(jnp.exp(-jnp.abs(d)))


def _smooth_l1(x, y):
    d = jnp.abs(x - y)
    return jnp.where(d < 1.0, 0.5 * d * d, d - 0.5)


def _pass1_body(x_ref, tr_ref, tcl_ref, trn_ref, sin_ref, cos_ref, rad_ref,
                ce_ref, key_ref, sc_ref):
    b = pl.program_id(0)
    r = pl.program_id(1)

    l0 = x_ref[0, 0]
    l1 = x_ref[0, 1]
    t0 = x_ref[0, 2]
    t1 = x_ref[0, 3]
    sp = x_ref[0, 4]
    cp = x_ref[0, 5]
    rp = x_ref[0, 6]
    tr = tr_ref[0]
    tcl = tcl_ref[0]
    trn = trn_ref[0]
    sinm = sin_ref[0]
    cosm = cos_ref[0]
    radm = rad_ref[0]

    s = l1 - l0
    ce = _softplus(jnp.where(tr == 1, -s, s))
    posm = (tr * trn) != 0
    negm = ((1 - tr) * trn) != 0
    zero = jnp.zeros_like(ce)
    n_pos = jnp.sum(jnp.where(posm, 1.0, 0.0))
    loss_pos = jnp.sum(jnp.where(posm, ce, zero))
    n_negav = jnp.sum(jnp.where(negm, 1.0, 0.0))
    sum_neg = jnp.sum(jnp.where(negm, ce, zero))
    ce_ref[0] = jnp.where(negm, ce, jnp.full_like(ce, -1.0))
    kbits = lax.shift_right_logical(lax.bitcast_convert_type(ce, jnp.int32), 7)
    key_ref[0] = jnp.where(negm, kbits, jnp.full_like(kbits, _TRASH))

    st = t1 - t0
    ce_t = _softplus(jnp.where(tcl == 1, -st, st))
    ttmm = (trn * tr) != 0
    ttm_cnt = jnp.sum(jnp.where(ttmm, 1.0, 0.0))
    sum_tcl = jnp.sum(jnp.where(ttmm, ce_t, zero))

    sel = tcl != 0
    m_cnt = jnp.sum(jnp.where(sel, 1.0, 0.0))
    tct_cnt = jnp.sum(jnp.where((trn * tcl) != 0, 1.0, 0.0))
    scale = jnp.sqrt(1.0 / (sp * sp + cp * cp))
    s_rad = jnp.sum(jnp.where(sel, _smooth_l1(rp / radm, jnp.ones_like(rp)), zero))
    s_sin = jnp.sum(jnp.where(sel, _smooth_l1(sp * scale, sinm), zero))
    s_cos = jnp.sum(jnp.where(sel, _smooth_l1(cp * scale, cosm), zero))

    vals = [n_pos, loss_pos, n_negav, sum_neg, ttm_cnt, sum_tcl,
            m_cnt, tct_cnt, s_rad, s_sin, s_cos]
    first = (b == 0) & (r == 0)

    @pl.when(first)
    def _():
        for i, v in enumerate(vals):
            sc_ref[0, i] = v

    @pl.when(jnp.logical_not(first))
    def _():
        for i, v in enumerate(vals):
            sc_ref[0, i] = sc_ref[0, i] + v


def _run_pass1(x, tr, tcl, trn, sinm, cosm, radm):
    m_spec = pl.BlockSpec((1, _RB, _W), lambda b, r: (b, r, 0))
    w_spec = pl.BlockSpec((1, _RB, _W), lambda b, r: (b * 4 + r // 2, r % 2, 0))
    return pl.pallas_call(
        _pass1_body,
        grid=(_GB, _GR),
        in_specs=[
            pl.BlockSpec((1, 7, _RB, _W), lambda b, r: (b, 0, r, 0)),
            m_spec, m_spec, m_spec, m_spec, m_spec, m_spec,
        ],
        out_specs=[
            w_spec,
            w_spec,
            pl.BlockSpec((1, 16), lambda b, r: (0, 0),
                         memory_space=pltpu.SMEM),
        ],
        out_shape=[
            jax.ShapeDtypeStruct((NW, _WR, _W), jnp.float32),
            jax.ShapeDtypeStruct((NW, _WR, _W), jnp.int32),
            jax.ShapeDtypeStruct((1, 16), jnp.float32),
        ],
    )(x, tr, tcl, trn, sinm, cosm, radm)


def _zero_hist(cnt_v):
    zeros = jnp.zeros((16,), jnp.float32)

    def zbody(i, c):
        for u in range(8):
            cnt_v[pl.ds(i * 128 + u * 16, 16)] = zeros
        return c

    lax.fori_loop(0, NB // 128, zbody, 0)


def _sc_hist2_body(key_hbm, cnt_hbm, buf, cnt_v):
    wid = lax.axis_index("s") * 2 + lax.axis_index("c")
    pltpu.sync_copy(key_hbm.at[wid], buf)
    _zero_hist(cnt_v)
    ones = jnp.ones((16,), jnp.float32)
    trash = jnp.full((16,), _TRASH, jnp.int32)

    def row(i, c):
        for u in range(_W // 16):
            k = buf[i, pl.ds(u * 16, 16)]
            idx = lax.shift_right_logical(k, 12)
            plsc.addupdate_scatter(cnt_v, [idx], ones, mask=k < trash)
        return c

    lax.fori_loop(0, _WR, row, 0)
    pltpu.sync_copy(cnt_v, cnt_hbm.at[wid])


def _sc_hist3_body(key_hbm, bnd_hbm, cnt_hbm, buf, cnt_v, bnd_v):
    wid = lax.axis_index("s") * 2 + lax.axis_index("c")
    pltpu.sync_copy(key_hbm.at[wid], buf)
    pltpu.sync_copy(bnd_hbm, bnd_v)
    bsel = bnd_v[0, pl.ds(0, 16)].astype(jnp.int32)
    _zero_hist(cnt_v)
    ones = jnp.ones((16,), jnp.float32)
    low = jnp.full((16,), NB - 1, jnp.int32)

    def row(i, c):
        for u in range(_W // 16):
            k = buf[i, pl.ds(u * 16, 16)]
            coarse = lax.shift_right_logical(k, 12)
            idx = jnp.bitwise_and(k, low)
            plsc.addupdate_scatter(cnt_v, [idx], ones, mask=coarse == bsel)
        return c

    lax.fori_loop(0, _WR, row, 0)
    pltpu.sync_copy(cnt_v, cnt_hbm.at[wid])


@functools.cache
def _get_sc_hist(level):
    if level == 2:
        body = _sc_hist2_body
        scratch = [
            pltpu.VMEM((_WR, _W), jnp.int32),
            pltpu.VMEM((NB,), jnp.float32),
        ]
    else:
        body = _sc_hist3_body
        scratch = [
            pltpu.VMEM((_WR, _W), jnp.int32),
            pltpu.VMEM((NB,), jnp.float32),
            pltpu.VMEM((8, 128), jnp.float32),
        ]
    return pl.kernel(
        body,
        out_type=jax.ShapeDtypeStruct((NW, NB), jnp.float32),
        mesh=plsc.VectorSubcoreMesh(core_axis_name="c", subcore_axis_name="s"),
        compiler_params=pltpu.CompilerParams(needs_layout_passes=False),
        scratch_types=scratch,
    )


def _suffix_sum(x):
    s = x
    off = 1
    while off < NB:
        s = s + jnp.concatenate(
            [s[:, off:], jnp.zeros((1, off), s.dtype)], axis=1)
        off *= 2
    return s


def _select_body(sc_ref, cnt_ref, bnd_ref, car_ref):
    cnt = jnp.sum(cnt_ref[...], axis=0, keepdims=True)
    j = lax.broadcasted_iota(jnp.int32, (1, NB), 1).astype(jnp.float32)
    cnt = jnp.where(j < float(_TBIN), cnt, 0.0)
    n_pos = sc_ref[0, 0]
    avail = sc_ref[0, 2]
    k = jnp.where(n_pos > 0, jnp.minimum(avail, 3.0 * n_pos), 100.0)
    cge = _suffix_sum(cnt)
    b = jnp.max(jnp.where(cge >= k, j, -1.0))
    cnt_gt = jnp.sum(jnp.where(j > b, cnt, 0.0))
    k_rem = k - cnt_gt
    rows = lax.broadcasted_iota(jnp.int32, (8, 128), 0)
    bnd_ref[...] = jnp.where(rows == 0, b, 0.0)
    car_ref[0, 0] = k
    car_ref[0, 1] = k_rem
    car_ref[0, 2] = b
    for i in range(3, 8):
        car_ref[0, i] = 0.0


def _run_select(sc, cnt2):
    return pl.pallas_call(
        _select_body,
        in_specs=[
            pl.BlockSpec(memory_space=pltpu.SMEM),
            pl.BlockSpec(memory_space=pltpu.MemorySpace.VMEM),
        ],
        out_specs=[
            pl.BlockSpec(memory_space=pltpu.MemorySpace.VMEM),
            pl.BlockSpec(memory_space=pltpu.SMEM),
        ],
        out_shape=[
            jax.ShapeDtypeStruct((8, 128), jnp.float32),
            jax.ShapeDtypeStruct((1, 8), jnp.float32),
        ],
    )(sc, cnt2)


def _select2_body(car_ref, cnt_ref, car2_ref):
    cnt = jnp.sum(cnt_ref[...], axis=0, keepdims=True)
    k_rem = car_ref[0, 1]
    cge = _suffix_sum(cnt)
    j = lax.broadcasted_iota(jnp.int32, (1, NB), 1).astype(jnp.float32)
    b3 = jnp.max(jnp.where(cge >= k_rem, j, -1.0))
    cnt_gt3 = jnp.sum(jnp.where(j > b3, cnt, 0.0))
    k3 = k_rem - cnt_gt3
    cnt_eq3 = jnp.sum(jnp.where(j == b3, cnt, 0.0))
    car2_ref[0, 0] = car_ref[0, 0]
    car2_ref[0, 1] = car_ref[0, 2]
    car2_ref[0, 2] = b3
    car2_ref[0, 3] = k3
    car2_ref[0, 4] = cnt_eq3
    for i in range(5, 8):
        car2_ref[0, i] = 0.0


def _run_select2(carry, cnt3):
    return pl.pallas_call(
        _select2_body,
        in_specs=[
            pl.BlockSpec(memory_space=pltpu.SMEM),
            pl.BlockSpec(memory_space=pltpu.MemorySpace.VMEM),
        ],
        out_specs=pl.BlockSpec(memory_space=pltpu.SMEM),
        out_shape=jax.ShapeDtypeStruct((1, 8), jnp.float32),
    )(carry, cnt3)


def _final_body(sc_ref, car2_ref, ce_ref, out_ref):
    w = pl.program_id(0)
    b_i = car2_ref[0, 1].astype(jnp.int32)
    b3_i = car2_ref[0, 2].astype(jnp.int32)
    kthr = b_i * NB + b3_i

    v = ce_ref[0]
    valid = v >= 0.0
    key = lax.shift_right_logical(lax.bitcast_convert_type(v, jnp.int32), 7)
    zero = jnp.zeros_like(v)
    s_gt = jnp.sum(jnp.where(valid & (key > kthr), v, zero))
    s_eq = jnp.sum(jnp.where(valid & (key == kthr), v, zero))

    @pl.when(w == 0)
    def _():
        out_ref[0, 5] = s_gt
        out_ref[0, 6] = s_eq

    @pl.when(w != 0)
    def _():
        out_ref[0, 5] = out_ref[0, 5] + s_gt
        out_ref[0, 6] = out_ref[0, 6] + s_eq

    @pl.when(w == NW - 1)
    def _():
        k = car2_ref[0, 0]
        k3 = car2_ref[0, 3]
        cnt_eq3 = car2_ref[0, 4]
        mean3 = out_ref[0, 6] / jnp.maximum(cnt_eq3, 1.0)
        topk = out_ref[0, 5] + k3 * mean3

        n_pos = sc_ref[0, 0]
        loss_pos = sc_ref[0, 1]
        avail = sc_ref[0, 2]
        sum_neg_tot = sc_ref[0, 3]
        loss_neg = jnp.where(k >= avail, sum_neg_tot,
                             jnp.where(k <= 0.0, 0.0, topk))
        loss_tr = (loss_pos + loss_neg) / (n_pos + k)

        ttm_cnt = sc_ref[0, 4]
        sum_tcl = sc_ref[0, 5]
        m_cnt = sc_ref[0, 6]
        tct_cnt = sc_ref[0, 7]
        loss_tcl = jnp.where(ttm_cnt > 0,
                             sum_tcl / jnp.maximum(ttm_cnt, 1.0), 0.0)
        gd = jnp.maximum(m_cnt, 1.0)
        out_ref[0, 0] = loss_tr
        out_ref[0, 1] = loss_tcl
        out_ref[0, 2] = jnp.where(tct_cnt > 0, sc_ref[0, 8] / gd, 0.0)
        out_ref[0, 3] = jnp.where(tct_cnt > 0, sc_ref[0, 9] / gd, 0.0)
        out_ref[0, 4] = jnp.where(tct_cnt > 0, sc_ref[0, 10] / gd, 0.0)


def _run_final(sc, carry2, ce_neg):
    return pl.pallas_call(
        _final_body,
        grid=(NW,),
        in_specs=[
            pl.BlockSpec((1, 16), lambda w: (0, 0), memory_space=pltpu.SMEM),
            pl.BlockSpec((1, 8), lambda w: (0, 0), memory_space=pltpu.SMEM),
            pl.BlockSpec((1, _WR, _W), lambda w: (w, 0, 0)),
        ],
        out_specs=pl.BlockSpec((1, 8), lambda w: (0, 0),
                               memory_space=pltpu.SMEM),
        out_shape=jax.ShapeDtypeStruct((1, 8), jnp.float32),
    )(sc, carry2, ce_neg)


def kernel(input, tr_mask, tcl_mask, sin_map, cos_map, radii_map, train_mask):
    ce_neg, key, sc = _run_pass1(input, tr_mask, tcl_mask, train_mask,
                                 sin_map, cos_map, radii_map)
    cnt2 = _get_sc_hist(2)(key)
    bnd3, carry = _run_select(sc, cnt2)
    cnt3 = _get_sc_hist(3)(key, bnd3)
    carry2 = _run_select2(carry, cnt3)
    out = _run_final(sc, carry2, ce_neg)
    return (out[0, 0], out[0, 1], out[0, 2], out[0, 3], out[0, 4])


# speedup vs baseline: 20.6645x; 1.0031x over previous
import functools

import jax
import jax.numpy as jnp
from jax import lax
from jax.experimental import pallas as pl
from jax.experimental.pallas import tpu as pltpu
from jax.experimental.pallas import tpu_sc as plsc

N = 8 * 512 * 512
NB = 4096
NW = 32
_H, _W = 512, 512
_RB = 64
_GB, _GR = 8, _H // _RB
_WR = 128
_TRASH = 4095 << 12
_TBIN = 4080


def _softplus(d):
    return jnp.maximum(d, 0.0) + jnp.log1p(jnp.exp(-jnp.abs(d)))


def _smooth_l1(x, y):
    d = jnp.abs(x - y)
    return jnp.where(d < 1.0, 0.5 * d * d, d - 0.5)


def _pass1_body(x_ref, tr_ref, tcl_ref, trn_ref, sin_ref, cos_ref, rad_ref,
                ce_ref, key_ref, sc_ref):
    b = pl.program_id(0)
    r = pl.program_id(1)

    l0 = x_ref[0, 0]
    l1 = x_ref[0, 1]
    t0 = x_ref[0, 2]
    t1 = x_ref[0, 3]
    sp = x_ref[0, 4]
    cp = x_ref[0, 5]
    rp = x_ref[0, 6]
    tr = tr_ref[0]
    tcl = tcl_ref[0]
    trn = trn_ref[0]
    sinm = sin_ref[0]
    cosm = cos_ref[0]
    radm = rad_ref[0]

    s = l1 - l0
    ce = _softplus(jnp.where(tr == 1, -s, s))
    posm = (tr * trn) != 0
    negm = ((1 - tr) * trn) != 0
    zero = jnp.zeros_like(ce)
    n_pos = jnp.sum(jnp.where(posm, 1.0, 0.0))
    loss_pos = jnp.sum(jnp.where(posm, ce, zero))
    n_negav = jnp.sum(jnp.where(negm, 1.0, 0.0))
    sum_neg = jnp.sum(jnp.where(negm, ce, zero))
    ce_ref[0] = jnp.where(negm, ce, jnp.full_like(ce, -1.0))
    kbits = lax.shift_right_logical(lax.bitcast_convert_type(ce, jnp.int32), 7)
    key_ref[0] = jnp.where(negm, kbits, jnp.full_like(kbits, _TRASH))

    st = t1 - t0
    ce_t = _softplus(jnp.where(tcl == 1, -st, st))
    ttmm = (trn * tr) != 0
    ttm_cnt = jnp.sum(jnp.where(ttmm, 1.0, 0.0))
    sum_tcl = jnp.sum(jnp.where(ttmm, ce_t, zero))

    sel = tcl != 0
    m_cnt = jnp.sum(jnp.where(sel, 1.0, 0.0))
    tct_cnt = jnp.sum(jnp.where((trn * tcl) != 0, 1.0, 0.0))
    scale = jnp.sqrt(1.0 / (sp * sp + cp * cp))
    s_rad = jnp.sum(jnp.where(sel, _smooth_l1(rp / radm, jnp.ones_like(rp)), zero))
    s_sin = jnp.sum(jnp.where(sel, _smooth_l1(sp * scale, sinm), zero))
    s_cos = jnp.sum(jnp.where(sel, _smooth_l1(cp * scale, cosm), zero))

    vals = [n_pos, loss_pos, n_negav, sum_neg, ttm_cnt, sum_tcl,
            m_cnt, tct_cnt, s_rad, s_sin, s_cos]
    first = (b == 0) & (r == 0)

    @pl.when(first)
    def _():
        for i, v in enumerate(vals):
            sc_ref[0, i] = v

    @pl.when(jnp.logical_not(first))
    def _():
        for i, v in enumerate(vals):
            sc_ref[0, i] = sc_ref[0, i] + v


def _run_pass1(x, tr, tcl, trn, sinm, cosm, radm):
    m_spec = pl.BlockSpec((1, _RB, _W), lambda b, r: (b, r, 0))
    w_spec = pl.BlockSpec((1, _RB, _W), lambda b, r: (b * 4 + r // 2, r % 2, 0))
    return pl.pallas_call(
        _pass1_body,
        grid=(_GB, _GR),
        in_specs=[
            pl.BlockSpec((1, 7, _RB, _W), lambda b, r: (b, 0, r, 0)),
            m_spec, m_spec, m_spec, m_spec, m_spec, m_spec,
        ],
        out_specs=[
            w_spec,
            w_spec,
            pl.BlockSpec((1, 16), lambda b, r: (0, 0),
                         memory_space=pltpu.SMEM),
        ],
        out_shape=[
            jax.ShapeDtypeStruct((NW, _WR, _W), jnp.float32),
            jax.ShapeDtypeStruct((NW, _WR, _W), jnp.int32),
            jax.ShapeDtypeStruct((1, 16), jnp.float32),
        ],
    )(x, tr, tcl, trn, sinm, cosm, radm)


def _zero_hist(cnt_v):
    zeros = jnp.zeros((16,), jnp.float32)

    def zbody(i, c):
        for u in range(8):
            cnt_v[pl.ds(i * 128 + u * 16, 16)] = zeros
        return c

    lax.fori_loop(0, NB // 128, zbody, 0)


_CH = 8
_RC = _WR // _CH


def _dbuf_scan(key_hbm, wid, buf, sems, process):
    pend = pltpu.async_copy(
        key_hbm.at[wid, pl.ds(0, _RC)], buf.at[0], sems[0])
    for c in range(_CH):
        nxt = None
        if c + 1 < _CH:
            nxt = pltpu.async_copy(
                key_hbm.at[wid, pl.ds((c + 1) * _RC, _RC)],
                buf.at[(c + 1) % 2], sems[(c + 1) % 2])
        pend.wait()
        process(c % 2)
        pend = nxt


def _sc_hist2_body(key_hbm, cnt_hbm, buf, cnt_v, sem0, sem1):
    wid = lax.axis_index("s") * 2 + lax.axis_index("c")
    _zero_hist(cnt_v)
    ones = jnp.ones((16,), jnp.float32)
    trash = jnp.full((16,), _TRASH, jnp.int32)

    def process(par):
        def row(i, c):
            for u in range(_W // 16):
                k = buf[par, i, pl.ds(u * 16, 16)]
                idx = lax.shift_right_logical(k, 12)
                plsc.addupdate_scatter(cnt_v, [idx], ones, mask=k < trash)
            return c
        lax.fori_loop(0, _RC, row, 0)

    _dbuf_scan(key_hbm, wid, buf, (sem0, sem1), process)
    pltpu.sync_copy(cnt_v, cnt_hbm.at[wid])


def _sc_hist3_body(key_hbm, bnd_hbm, cnt_hbm, buf, cnt_v, bnd_v, sem0, sem1):
    wid = lax.axis_index("s") * 2 + lax.axis_index("c")
    pltpu.sync_copy(bnd_hbm, bnd_v)
    bsel = bnd_v[0, pl.ds(0, 16)].astype(jnp.int32)
    _zero_hist(cnt_v)
    ones = jnp.ones((16,), jnp.float32)
    low = jnp.full((16,), NB - 1, jnp.int32)

    def process(par):
        def row(i, c):
            for u in range(_W // 16):
                k = buf[par, i, pl.ds(u * 16, 16)]
                coarse = lax.shift_right_logical(k, 12)
                idx = jnp.bitwise_and(k, low)
                plsc.addupdate_scatter(cnt_v, [idx], ones,
                                       mask=coarse == bsel)
            return c
        lax.fori_loop(0, _RC, row, 0)

    _dbuf_scan(key_hbm, wid, buf, (sem0, sem1), process)
    pltpu.sync_copy(cnt_v, cnt_hbm.at[wid])


@functools.cache
def _get_sc_hist(level):
    if level == 2:
        body = _sc_hist2_body
        scratch = [
            pltpu.VMEM((2, _RC, _W), jnp.int32),
            pltpu.VMEM((NB,), jnp.float32),
            pltpu.SemaphoreType.DMA,
            pltpu.SemaphoreType.DMA,
        ]
    else:
        body = _sc_hist3_body
        scratch = [
            pltpu.VMEM((2, _RC, _W), jnp.int32),
            pltpu.VMEM((NB,), jnp.float32),
            pltpu.VMEM((8, 128), jnp.float32),
            pltpu.SemaphoreType.DMA,
            pltpu.SemaphoreType.DMA,
        ]
    return pl.kernel(
        body,
        out_type=jax.ShapeDtypeStruct((NW, NB), jnp.float32),
        mesh=plsc.VectorSubcoreMesh(core_axis_name="c", subcore_axis_name="s"),
        compiler_params=pltpu.CompilerParams(needs_layout_passes=False),
        scratch_types=scratch,
    )


def _suffix_sum(x):
    s = x
    off = 1
    while off < NB:
        s = s + jnp.concatenate(
            [s[:, off:], jnp.zeros((1, off), s.dtype)], axis=1)
        off *= 2
    return s


def _select_body(sc_ref, cnt_ref, bnd_ref, car_ref):
    cnt = jnp.sum(cnt_ref[...], axis=0, keepdims=True)
    j = lax.broadcasted_iota(jnp.int32, (1, NB), 1).astype(jnp.float32)
    cnt = jnp.where(j < float(_TBIN), cnt, 0.0)
    n_pos = sc_ref[0, 0]
    avail = sc_ref[0, 2]
    k = jnp.where(n_pos > 0, jnp.minimum(avail, 3.0 * n_pos), 100.0)
    cge = _suffix_sum(cnt)
    b = jnp.max(jnp.where(cge >= k, j, -1.0))
    cnt_gt = jnp.sum(jnp.where(j > b, cnt, 0.0))
    k_rem = k - cnt_gt
    rows = lax.broadcasted_iota(jnp.int32, (8, 128), 0)
    bnd_ref[...] = jnp.where(rows == 0, b, 0.0)
    car_ref[0, 0] = k
    car_ref[0, 1] = k_rem
    car_ref[0, 2] = b
    for i in range(3, 8):
        car_ref[0, i] = 0.0


def _run_select(sc, cnt2):
    return pl.pallas_call(
        _select_body,
        in_specs=[
            pl.BlockSpec(memory_space=pltpu.SMEM),
            pl.BlockSpec(memory_space=pltpu.MemorySpace.VMEM),
        ],
        out_specs=[
            pl.BlockSpec(memory_space=pltpu.MemorySpace.VMEM),
            pl.BlockSpec(memory_space=pltpu.SMEM),
        ],
        out_shape=[
            jax.ShapeDtypeStruct((8, 128), jnp.float32),
            jax.ShapeDtypeStruct((1, 8), jnp.float32),
        ],
    )(sc, cnt2)


def _select2_body(car_ref, cnt_ref, car2_ref):
    cnt = jnp.sum(cnt_ref[...], axis=0, keepdims=True)
    k_rem = car_ref[0, 1]
    cge = _suffix_sum(cnt)
    j = lax.broadcasted_iota(jnp.int32, (1, NB), 1).astype(jnp.float32)
    b3 = jnp.max(jnp.where(cge >= k_rem, j, -1.0))
    cnt_gt3 = jnp.sum(jnp.where(j > b3, cnt, 0.0))
    k3 = k_rem - cnt_gt3
    cnt_eq3 = jnp.sum(jnp.where(j == b3, cnt, 0.0))
    car2_ref[0, 0] = car_ref[0, 0]
    car2_ref[0, 1] = car_ref[0, 2]
    car2_ref[0, 2] = b3
    car2_ref[0, 3] = k3
    car2_ref[0, 4] = cnt_eq3
    for i in range(5, 8):
        car2_ref[0, i] = 0.0


def _run_select2(carry, cnt3):
    return pl.pallas_call(
        _select2_body,
        in_specs=[
            pl.BlockSpec(memory_space=pltpu.SMEM),
            pl.BlockSpec(memory_space=pltpu.MemorySpace.VMEM),
        ],
        out_specs=pl.BlockSpec(memory_space=pltpu.SMEM),
        out_shape=jax.ShapeDtypeStruct((1, 8), jnp.float32),
    )(carry, cnt3)


def _final_body(sc_ref, car2_ref, ce_ref, out_ref):
    w = pl.program_id(0)
    b_i = car2_ref[0, 1].astype(jnp.int32)
    b3_i = car2_ref[0, 2].astype(jnp.int32)
    kthr = b_i * NB + b3_i

    v = ce_ref[0]
    valid = v >= 0.0
    key = lax.shift_right_logical(lax.bitcast_convert_type(v, jnp.int32), 7)
    zero = jnp.zeros_like(v)
    s_gt = jnp.sum(jnp.where(valid & (key > kthr), v, zero))
    s_eq = jnp.sum(jnp.where(valid & (key == kthr), v, zero))

    @pl.when(w == 0)
    def _():
        out_ref[0, 5] = s_gt
        out_ref[0, 6] = s_eq

    @pl.when(w != 0)
    def _():
        out_ref[0, 5] = out_ref[0, 5] + s_gt
        out_ref[0, 6] = out_ref[0, 6] + s_eq

    @pl.when(w == NW - 1)
    def _():
        k = car2_ref[0, 0]
        k3 = car2_ref[0, 3]
        cnt_eq3 = car2_ref[0, 4]
        mean3 = out_ref[0, 6] / jnp.maximum(cnt_eq3, 1.0)
        topk = out_ref[0, 5] + k3 * mean3

        n_pos = sc_ref[0, 0]
        loss_pos = sc_ref[0, 1]
        avail = sc_ref[0, 2]
        sum_neg_tot = sc_ref[0, 3]
        loss_neg = jnp.where(k >= avail, sum_neg_tot,
                             jnp.where(k <= 0.0, 0.0, topk))
        loss_tr = (loss_pos + loss_neg) / (n_pos + k)

        ttm_cnt = sc_ref[0, 4]
        sum_tcl = sc_ref[0, 5]
        m_cnt = sc_ref[0, 6]
        tct_cnt = sc_ref[0, 7]
        loss_tcl = jnp.where(ttm_cnt > 0,
                             sum_tcl / jnp.maximum(ttm_cnt, 1.0), 0.0)
        gd = jnp.maximum(m_cnt, 1.0)
        out_ref[0, 0] = loss_tr
        out_ref[0, 1] = loss_tcl
        out_ref[0, 2] = jnp.where(tct_cnt > 0, sc_ref[0, 8] / gd, 0.0)
        out_ref[0, 3] = jnp.where(tct_cnt > 0, sc_ref[0, 9] / gd, 0.0)
        out_ref[0, 4] = jnp.where(tct_cnt > 0, sc_ref[0, 10] / gd, 0.0)


def _run_final(sc, carry2, ce_neg):
    return pl.pallas_call(
        _final_body,
        grid=(NW,),
        in_specs=[
            pl.BlockSpec((1, 16), lambda w: (0, 0), memory_space=pltpu.SMEM),
            pl.BlockSpec((1, 8), lambda w: (0, 0), memory_space=pltpu.SMEM),
            pl.BlockSpec((1, _WR, _W), lambda w: (w, 0, 0)),
        ],
        out_specs=pl.BlockSpec((1, 8), lambda w: (0, 0),
                               memory_space=pltpu.SMEM),
        out_shape=jax.ShapeDtypeStruct((1, 8), jnp.float32),
    )(sc, carry2, ce_neg)


def kernel(input, tr_mask, tcl_mask, sin_map, cos_map, radii_map, train_mask):
    ce_neg, key, sc = _run_pass1(input, tr_mask, tcl_mask, train_mask,
                                 sin_map, cos_map, radii_map)
    cnt2 = _get_sc_hist(2)(key)
    bnd3, carry = _run_select(sc, cnt2)
    cnt3 = _get_sc_hist(3)(key, bnd3)
    carry2 = _run_select2(carry, cnt3)
    out = _run_final(sc, carry2, ce_neg)
    return (out[0, 0], out[0, 1], out[0, 2], out[0, 3], out[0, 4])


# speedup vs baseline: 20.8178x; 1.0074x over previous
import functools

import jax
import jax.numpy as jnp
from jax import lax
from jax.experimental import pallas as pl
from jax.experimental.pallas import tpu as pltpu
from jax.experimental.pallas import tpu_sc as plsc

N = 8 * 512 * 512
NB = 4096
NW = 32
_H, _W = 512, 512
_RB = 64
_GB, _GR = 8, _H // _RB
_WR = 128
_TRASH = 4095 << 12
_TBIN = 4080


def _softplus(d):
    return jnp.maximum(d, 0.0) + jnp.log1p(jnp.exp(-jnp.abs(d)))


def _smooth_l1(x, y):
    d = jnp.abs(x - y)
    return jnp.where(d < 1.0, 0.5 * d * d, d - 0.5)


def _pass1_body(x_ref, tr_ref, tcl_ref, trn_ref, sin_ref, cos_ref, rad_ref,
                ce_ref, key_ref, sc_ref):
    b = pl.program_id(0)
    r = pl.program_id(1)

    l0 = x_ref[0, 0]
    l1 = x_ref[0, 1]
    t0 = x_ref[0, 2]
    t1 = x_ref[0, 3]
    sp = x_ref[0, 4]
    cp = x_ref[0, 5]
    rp = x_ref[0, 6]
    tr = tr_ref[0]
    tcl = tcl_ref[0]
    trn = trn_ref[0]
    sinm = sin_ref[0]
    cosm = cos_ref[0]
    radm = rad_ref[0]

    s = l1 - l0
    ce = _softplus(jnp.where(tr == 1, -s, s))
    posm = (tr * trn) != 0
    negm = ((1 - tr) * trn) != 0
    zero = jnp.zeros_like(ce)
    n_pos = jnp.sum(jnp.where(posm, 1.0, 0.0))
    loss_pos = jnp.sum(jnp.where(posm, ce, zero))
    n_negav = jnp.sum(jnp.where(negm, 1.0, 0.0))
    sum_neg = jnp.sum(jnp.where(negm, ce, zero))
    ce_ref[0] = jnp.where(negm, ce, jnp.full_like(ce, -1.0))
    kbits = lax.shift_right_logical(lax.bitcast_convert_type(ce, jnp.int32), 7)
    key_ref[0] = jnp.where(negm, kbits, jnp.full_like(kbits, _TRASH))

    st = t1 - t0
    ce_t = _softplus(jnp.where(tcl == 1, -st, st))
    ttmm = (trn * tr) != 0
    ttm_cnt = jnp.sum(jnp.where(ttmm, 1.0, 0.0))
    sum_tcl = jnp.sum(jnp.where(ttmm, ce_t, zero))

    sel = tcl != 0
    m_cnt = jnp.sum(jnp.where(sel, 1.0, 0.0))
    tct_cnt = jnp.sum(jnp.where((trn * tcl) != 0, 1.0, 0.0))
    scale = jnp.sqrt(1.0 / (sp * sp + cp * cp))
    s_rad = jnp.sum(jnp.where(sel, _smooth_l1(rp / radm, jnp.ones_like(rp)), zero))
    s_sin = jnp.sum(jnp.where(sel, _smooth_l1(sp * scale, sinm), zero))
    s_cos = jnp.sum(jnp.where(sel, _smooth_l1(cp * scale, cosm), zero))

    vals = [n_pos, loss_pos, n_negav, sum_neg, ttm_cnt, sum_tcl,
            m_cnt, tct_cnt, s_rad, s_sin, s_cos]
    first = (b == 0) & (r == 0)

    @pl.when(first)
    def _():
        for i, v in enumerate(vals):
            sc_ref[0, i] = v

    @pl.when(jnp.logical_not(first))
    def _():
        for i, v in enumerate(vals):
            sc_ref[0, i] = sc_ref[0, i] + v


def _run_pass1(x, tr, tcl, trn, sinm, cosm, radm):
    m_spec = pl.BlockSpec((1, _RB, _W), lambda b, r: (b, r, 0))
    w_spec = pl.BlockSpec((1, _RB, _W), lambda b, r: (b * 4 + r // 2, r % 2, 0))
    return pl.pallas_call(
        _pass1_body,
        grid=(_GB, _GR),
        in_specs=[
            pl.BlockSpec((1, 7, _RB, _W), lambda b, r: (b, 0, r, 0)),
            m_spec, m_spec, m_spec, m_spec, m_spec, m_spec,
        ],
        out_specs=[
            w_spec,
            w_spec,
            pl.BlockSpec((1, 16), lambda b, r: (0, 0),
                         memory_space=pltpu.SMEM),
        ],
        out_shape=[
            jax.ShapeDtypeStruct((NW, _WR, _W), jnp.float32),
            jax.ShapeDtypeStruct((NW, _WR, _W), jnp.int32),
            jax.ShapeDtypeStruct((1, 16), jnp.float32),
        ],
    )(x, tr, tcl, trn, sinm, cosm, radm)


def _zero_hist(cnt_v):
    zeros = jnp.zeros((16,), jnp.float32)

    def zbody(i, c):
        for u in range(8):
            cnt_v[pl.ds(i * 128 + u * 16, 16)] = zeros
        return c

    lax.fori_loop(0, NB // 128, zbody, 0)


_CH = 8
_RC = _WR // _CH


def _dbuf_scan(key_hbm, wid, buf, sems, process):
    pend = pltpu.async_copy(
        key_hbm.at[wid, pl.ds(0, _RC)], buf.at[0], sems[0])
    for c in range(_CH):
        nxt = None
        if c + 1 < _CH:
            nxt = pltpu.async_copy(
                key_hbm.at[wid, pl.ds((c + 1) * _RC, _RC)],
                buf.at[(c + 1) % 2], sems[(c + 1) % 2])
        pend.wait()
        process(c % 2)
        pend = nxt


def _sc_hist2_body(key_hbm, cnt_hbm, buf, cnt_v, sem0, sem1):
    wid = lax.axis_index("s") * 2 + lax.axis_index("c")
    _zero_hist(cnt_v)
    ones = jnp.ones((16,), jnp.float32)
    trash = jnp.full((16,), _TRASH, jnp.int32)

    def process(par):
        def row(i, c):
            for u in range(_W // 16):
                k = buf[par, i, pl.ds(u * 16, 16)]
                idx = lax.shift_right_logical(k, 12)
                plsc.addupdate_scatter(cnt_v, [idx], ones, mask=k < trash)
            return c
        lax.fori_loop(0, _RC, row, 0)

    _dbuf_scan(key_hbm, wid, buf, (sem0, sem1), process)
    pltpu.sync_copy(cnt_v, cnt_hbm.at[wid])


def _sc_hist3_body(key_hbm, bnd_hbm, cnt_hbm, buf, cnt_v, bnd_v, sem0, sem1):
    wid = lax.axis_index("s") * 2 + lax.axis_index("c")
    pltpu.sync_copy(bnd_hbm, bnd_v)
    bsel = bnd_v[0, pl.ds(0, 16)].astype(jnp.int32)
    _zero_hist(cnt_v)
    ones = jnp.ones((16,), jnp.float32)
    low = jnp.full((16,), NB - 1, jnp.int32)

    def process(par):
        def row(i, c):
            for u in range(_W // 16):
                k = buf[par, i, pl.ds(u * 16, 16)]
                coarse = lax.shift_right_logical(k, 12)
                idx = jnp.bitwise_and(k, low)
                plsc.addupdate_scatter(cnt_v, [idx], ones,
                                       mask=coarse == bsel)
            return c
        lax.fori_loop(0, _RC, row, 0)

    _dbuf_scan(key_hbm, wid, buf, (sem0, sem1), process)
    pltpu.sync_copy(cnt_v, cnt_hbm.at[wid])


@functools.cache
def _get_sc_hist(level):
    if level == 2:
        body = _sc_hist2_body
        scratch = [
            pltpu.VMEM((2, _RC, _W), jnp.int32),
            pltpu.VMEM((NB,), jnp.float32),
            pltpu.SemaphoreType.DMA,
            pltpu.SemaphoreType.DMA,
        ]
    else:
        body = _sc_hist3_body
        scratch = [
            pltpu.VMEM((2, _RC, _W), jnp.int32),
            pltpu.VMEM((NB,), jnp.float32),
            pltpu.VMEM((8, 128), jnp.float32),
            pltpu.SemaphoreType.DMA,
            pltpu.SemaphoreType.DMA,
        ]
    return pl.kernel(
        body,
        out_type=jax.ShapeDtypeStruct((NW, NB), jnp.float32),
        mesh=plsc.VectorSubcoreMesh(core_axis_name="c", subcore_axis_name="s"),
        compiler_params=pltpu.CompilerParams(needs_layout_passes=False),
        scratch_types=scratch,
    )


def _suffix_sum(x):
    s = x
    off = 1
    while off < NB:
        s = s + jnp.concatenate(
            [s[:, off:], jnp.zeros((1, off), s.dtype)], axis=1)
        off *= 2
    return s


def _select_body(sc_ref, cnt_ref, bnd_ref, car_ref):
    cnt = jnp.sum(cnt_ref[...], axis=0, keepdims=True)
    j = lax.broadcasted_iota(jnp.int32, (1, NB), 1).astype(jnp.float32)
    cnt = jnp.where(j < float(_TBIN), cnt, 0.0)
    n_pos = sc_ref[0, 0]
    avail = sc_ref[0, 2]
    k = jnp.where(n_pos > 0, jnp.minimum(avail, 3.0 * n_pos), 100.0)
    cge = _suffix_sum(cnt)
    b = jnp.max(jnp.where(cge >= k, j, -1.0))
    cnt_gt = jnp.sum(jnp.where(j > b, cnt, 0.0))
    k_rem = k - cnt_gt
    rows = lax.broadcasted_iota(jnp.int32, (8, 128), 0)
    bnd_ref[...] = jnp.where(rows == 0, b, 0.0)
    car_ref[0, 0] = k
    car_ref[0, 1] = k_rem
    car_ref[0, 2] = b
    for i in range(3, 8):
        car_ref[0, i] = 0.0


def _run_select(sc, cnt2):
    return pl.pallas_call(
        _select_body,
        in_specs=[
            pl.BlockSpec(memory_space=pltpu.SMEM),
            pl.BlockSpec(memory_space=pltpu.MemorySpace.VMEM),
        ],
        out_specs=[
            pl.BlockSpec(memory_space=pltpu.MemorySpace.VMEM),
            pl.BlockSpec(memory_space=pltpu.SMEM),
        ],
        out_shape=[
            jax.ShapeDtypeStruct((8, 128), jnp.float32),
            jax.ShapeDtypeStruct((1, 8), jnp.float32),
        ],
    )(sc, cnt2)


def _final_body(sc_ref, car_ref, cnt_ref, ce_ref, out_ref):
    w = pl.program_id(0)

    @pl.when(w == 0)
    def _():
        cnt = jnp.sum(cnt_ref[...], axis=0, keepdims=True)
        k_rem = car_ref[0, 1]
        cge = _suffix_sum(cnt)
        j = lax.broadcasted_iota(jnp.int32, (1, NB), 1).astype(jnp.float32)
        b3 = jnp.max(jnp.where(cge >= k_rem, j, -1.0))
        cnt_gt3 = jnp.sum(jnp.where(j > b3, cnt, 0.0))
        out_ref[0, 7] = car_ref[0, 2] * float(NB) + b3
        out_ref[0, 8] = k_rem - cnt_gt3
        out_ref[0, 9] = jnp.sum(jnp.where(j == b3, cnt, 0.0))

    kthr = out_ref[0, 7].astype(jnp.int32)

    v = ce_ref[0]
    valid = v >= 0.0
    key = lax.shift_right_logical(lax.bitcast_convert_type(v, jnp.int32), 7)
    zero = jnp.zeros_like(v)
    s_gt = jnp.sum(jnp.where(valid & (key > kthr), v, zero))
    s_eq = jnp.sum(jnp.where(valid & (key == kthr), v, zero))

    @pl.when(w == 0)
    def _():
        out_ref[0, 5] = s_gt
        out_ref[0, 6] = s_eq

    @pl.when(w != 0)
    def _():
        out_ref[0, 5] = out_ref[0, 5] + s_gt
        out_ref[0, 6] = out_ref[0, 6] + s_eq

    @pl.when(w == NW - 1)
    def _():
        k = car_ref[0, 0]
        k3 = out_ref[0, 8]
        cnt_eq3 = out_ref[0, 9]
        mean3 = out_ref[0, 6] / jnp.maximum(cnt_eq3, 1.0)
        topk = out_ref[0, 5] + k3 * mean3

        n_pos = sc_ref[0, 0]
        loss_pos = sc_ref[0, 1]
        avail = sc_ref[0, 2]
        sum_neg_tot = sc_ref[0, 3]
        loss_neg = jnp.where(k >= avail, sum_neg_tot,
                             jnp.where(k <= 0.0, 0.0, topk))
        loss_tr = (loss_pos + loss_neg) / (n_pos + k)

        ttm_cnt = sc_ref[0, 4]
        sum_tcl = sc_ref[0, 5]
        m_cnt = sc_ref[0, 6]
        tct_cnt = sc_ref[0, 7]
        loss_tcl = jnp.where(ttm_cnt > 0,
                             sum_tcl / jnp.maximum(ttm_cnt, 1.0), 0.0)
        gd = jnp.maximum(m_cnt, 1.0)
        out_ref[0, 0] = loss_tr
        out_ref[0, 1] = loss_tcl
        out_ref[0, 2] = jnp.where(tct_cnt > 0, sc_ref[0, 8] / gd, 0.0)
        out_ref[0, 3] = jnp.where(tct_cnt > 0, sc_ref[0, 9] / gd, 0.0)
        out_ref[0, 4] = jnp.where(tct_cnt > 0, sc_ref[0, 10] / gd, 0.0)


def _run_final(sc, carry, cnt3, ce_neg):
    return pl.pallas_call(
        _final_body,
        grid=(NW,),
        in_specs=[
            pl.BlockSpec((1, 16), lambda w: (0, 0), memory_space=pltpu.SMEM),
            pl.BlockSpec((1, 8), lambda w: (0, 0), memory_space=pltpu.SMEM),
            pl.BlockSpec((NW, NB), lambda w: (0, 0),
                         memory_space=pltpu.MemorySpace.VMEM),
            pl.BlockSpec((1, _WR, _W), lambda w: (w, 0, 0)),
        ],
        out_specs=pl.BlockSpec((1, 16), lambda w: (0, 0),
                               memory_space=pltpu.SMEM),
        out_shape=jax.ShapeDtypeStruct((1, 16), jnp.float32),
    )(sc, carry, cnt3, ce_neg)


def kernel(input, tr_mask, tcl_mask, sin_map, cos_map, radii_map, train_mask):
    ce_neg, key, sc = _run_pass1(input, tr_mask, tcl_mask, train_mask,
                                 sin_map, cos_map, radii_map)
    cnt2 = _get_sc_hist(2)(key)
    bnd3, carry = _run_select(sc, cnt2)
    cnt3 = _get_sc_hist(3)(key, bnd3)
    out = _run_final(sc, carry, cnt3, ce_neg)
    return (out[0, 0], out[0, 1], out[0, 2], out[0, 3], out[0, 4])


# speedup vs baseline: 23.0991x; 1.1096x over previous
import functools

import jax
import jax.numpy as jnp
from jax import lax
from jax.experimental import pallas as pl
from jax.experimental.pallas import tpu as pltpu
from jax.experimental.pallas import tpu_sc as plsc

N = 8 * 512 * 512
NB = 4096
NW = 32
_H, _W = 512, 512
_RB = 128
_GB, _GR = 8, _H // _RB
_WR = 128
_TRASH = 4095 << 12
_TBIN = 4080


def _softplus(d):
    return jnp.maximum(d, 0.0) + jnp.log1p(jnp.exp(-jnp.abs(d)))


def _smooth_l1(x, y):
    d = jnp.abs(x - y)
    return jnp.where(d < 1.0, 0.5 * d * d, d - 0.5)


def _pass1_body(x_ref, tr_ref, tcl_ref, trn_ref, sin_ref, cos_ref, rad_ref,
                ce_ref, key_ref, sc_ref):
    b = pl.program_id(0)
    r = pl.program_id(1)

    l0 = x_ref[0, 0]
    l1 = x_ref[0, 1]
    t0 = x_ref[0, 2]
    t1 = x_ref[0, 3]
    sp = x_ref[0, 4]
    cp = x_ref[0, 5]
    rp = x_ref[0, 6]
    tr = tr_ref[0]
    tcl = tcl_ref[0]
    trn = trn_ref[0]
    sinm = sin_ref[0]
    cosm = cos_ref[0]
    radm = rad_ref[0]

    s = l1 - l0
    ce = _softplus(jnp.where(tr == 1, -s, s))
    posm = (tr * trn) != 0
    negm = ((1 - tr) * trn) != 0
    zero = jnp.zeros_like(ce)
    n_pos = jnp.sum(jnp.where(posm, 1.0, 0.0))
    loss_pos = jnp.sum(jnp.where(posm, ce, zero))
    n_negav = jnp.sum(jnp.where(negm, 1.0, 0.0))
    sum_neg = jnp.sum(jnp.where(negm, ce, zero))
    ce_ref[0] = jnp.where(negm, ce, jnp.full_like(ce, -1.0))
    kbits = lax.shift_right_logical(lax.bitcast_convert_type(ce, jnp.int32), 7)
    key_ref[0] = jnp.where(negm, kbits, jnp.full_like(kbits, _TRASH))

    st = t1 - t0
    ce_t = _softplus(jnp.where(tcl == 1, -st, st))
    ttmm = (trn * tr) != 0
    ttm_cnt = jnp.sum(jnp.where(ttmm, 1.0, 0.0))
    sum_tcl = jnp.sum(jnp.where(ttmm, ce_t, zero))

    sel = tcl != 0
    m_cnt = jnp.sum(jnp.where(sel, 1.0, 0.0))
    tct_cnt = jnp.sum(jnp.where((trn * tcl) != 0, 1.0, 0.0))
    scale = jnp.sqrt(1.0 / (sp * sp + cp * cp))
    s_rad = jnp.sum(jnp.where(sel, _smooth_l1(rp / radm, jnp.ones_like(rp)), zero))
    s_sin = jnp.sum(jnp.where(sel, _smooth_l1(sp * scale, sinm), zero))
    s_cos = jnp.sum(jnp.where(sel, _smooth_l1(cp * scale, cosm), zero))

    vals = [n_pos, loss_pos, n_negav, sum_neg, ttm_cnt, sum_tcl,
            m_cnt, tct_cnt, s_rad, s_sin, s_cos]
    first = (b == 0) & (r == 0)

    @pl.when(first)
    def _():
        for i, v in enumerate(vals):
            sc_ref[0, i] = v

    @pl.when(jnp.logical_not(first))
    def _():
        for i, v in enumerate(vals):
            sc_ref[0, i] = sc_ref[0, i] + v


def _run_pass1(x, tr, tcl, trn, sinm, cosm, radm):
    m_spec = pl.BlockSpec((1, _RB, _W), lambda b, r: (b, r, 0))
    w_spec = pl.BlockSpec((1, _RB, _W), lambda b, r: (b * _GR + r, 0, 0))
    return pl.pallas_call(
        _pass1_body,
        grid=(_GB, _GR),
        in_specs=[
            pl.BlockSpec((1, 7, _RB, _W), lambda b, r: (b, 0, r, 0)),
            m_spec, m_spec, m_spec, m_spec, m_spec, m_spec,
        ],
        out_specs=[
            w_spec,
            w_spec,
            pl.BlockSpec((1, 16), lambda b, r: (0, 0),
                         memory_space=pltpu.SMEM),
        ],
        out_shape=[
            jax.ShapeDtypeStruct((NW, _WR, _W), jnp.float32),
            jax.ShapeDtypeStruct((NW, _WR, _W), jnp.int32),
            jax.ShapeDtypeStruct((1, 16), jnp.float32),
        ],
    )(x, tr, tcl, trn, sinm, cosm, radm)


def _zero_hist(cnt_v):
    zeros = jnp.zeros((16,), jnp.float32)

    def zbody(i, c):
        for u in range(8):
            cnt_v[pl.ds(i * 128 + u * 16, 16)] = zeros
        return c

    lax.fori_loop(0, NB // 128, zbody, 0)


_CH = 8
_RC = _WR // _CH


def _dbuf_scan(key_hbm, wid, buf, sems, process):
    pend = pltpu.async_copy(
        key_hbm.at[wid, pl.ds(0, _RC)], buf.at[0], sems[0])
    for c in range(_CH):
        nxt = None
        if c + 1 < _CH:
            nxt = pltpu.async_copy(
                key_hbm.at[wid, pl.ds((c + 1) * _RC, _RC)],
                buf.at[(c + 1) % 2], sems[(c + 1) % 2])
        pend.wait()
        process(c % 2)
        pend = nxt


def _sc_hist2_body(key_hbm, cnt_hbm, buf, cnt_v, sem0, sem1):
    wid = lax.axis_index("s") * 2 + lax.axis_index("c")
    _zero_hist(cnt_v)
    ones = jnp.ones((16,), jnp.float32)
    trash = jnp.full((16,), _TRASH, jnp.int32)

    def process(par):
        def row(i, c):
            for u in range(_W // 16):
                k = buf[par, i, pl.ds(u * 16, 16)]
                idx = lax.shift_right_logical(k, 12)
                plsc.addupdate_scatter(cnt_v, [idx], ones, mask=k < trash)
            return c
        lax.fori_loop(0, _RC, row, 0)

    _dbuf_scan(key_hbm, wid, buf, (sem0, sem1), process)
    pltpu.sync_copy(cnt_v, cnt_hbm.at[wid])


def _sc_hist3_body(key_hbm, bnd_hbm, cnt_hbm, buf, cnt_v, bnd_v, sem0, sem1):
    wid = lax.axis_index("s") * 2 + lax.axis_index("c")
    pltpu.sync_copy(bnd_hbm, bnd_v)
    bsel = bnd_v[0, pl.ds(0, 16)].astype(jnp.int32)
    _zero_hist(cnt_v)
    ones = jnp.ones((16,), jnp.float32)
    low = jnp.full((16,), NB - 1, jnp.int32)

    def process(par):
        def row(i, c):
            for u in range(_W // 16):
                k = buf[par, i, pl.ds(u * 16, 16)]
                coarse = lax.shift_right_logical(k, 12)
                idx = jnp.bitwise_and(k, low)
                plsc.addupdate_scatter(cnt_v, [idx], ones,
                                       mask=coarse == bsel)
            return c
        lax.fori_loop(0, _RC, row, 0)

    _dbuf_scan(key_hbm, wid, buf, (sem0, sem1), process)
    pltpu.sync_copy(cnt_v, cnt_hbm.at[wid])


@functools.cache
def _get_sc_hist(level):
    if level == 2:
        body = _sc_hist2_body
        scratch = [
            pltpu.VMEM((2, _RC, _W), jnp.int32),
            pltpu.VMEM((NB,), jnp.float32),
            pltpu.SemaphoreType.DMA,
            pltpu.SemaphoreType.DMA,
        ]
    else:
        body = _sc_hist3_body
        scratch = [
            pltpu.VMEM((2, _RC, _W), jnp.int32),
            pltpu.VMEM((NB,), jnp.float32),
            pltpu.VMEM((8, 128), jnp.float32),
            pltpu.SemaphoreType.DMA,
            pltpu.SemaphoreType.DMA,
        ]
    return pl.kernel(
        body,
        out_type=jax.ShapeDtypeStruct((NW, NB), jnp.float32),
        mesh=plsc.VectorSubcoreMesh(core_axis_name="c", subcore_axis_name="s"),
        compiler_params=pltpu.CompilerParams(needs_layout_passes=False),
        scratch_types=scratch,
    )


def _suffix_sum(x):
    s = x
    off = 1
    while off < NB:
        s = s + jnp.concatenate(
            [s[:, off:], jnp.zeros((1, off), s.dtype)], axis=1)
        off *= 2
    return s


def _select_body(sc_ref, cnt_ref, bnd_ref, car_ref):
    cnt = jnp.sum(cnt_ref[...], axis=0, keepdims=True)
    j = lax.broadcasted_iota(jnp.int32, (1, NB), 1).astype(jnp.float32)
    cnt = jnp.where(j < float(_TBIN), cnt, 0.0)
    n_pos = sc_ref[0, 0]
    avail = sc_ref[0, 2]
    k = jnp.where(n_pos > 0, jnp.minimum(avail, 3.0 * n_pos), 100.0)
    cge = _suffix_sum(cnt)
    b = jnp.max(jnp.where(cge >= k, j, -1.0))
    cnt_gt = jnp.sum(jnp.where(j > b, cnt, 0.0))
    k_rem = k - cnt_gt
    rows = lax.broadcasted_iota(jnp.int32, (8, 128), 0)
    bnd_ref[...] = jnp.where(rows == 0, b, 0.0)
    car_ref[0, 0] = k
    car_ref[0, 1] = k_rem
    car_ref[0, 2] = b
    for i in range(3, 8):
        car_ref[0, i] = 0.0


def _run_select(sc, cnt2):
    return pl.pallas_call(
        _select_body,
        in_specs=[
            pl.BlockSpec(memory_space=pltpu.SMEM),
            pl.BlockSpec(memory_space=pltpu.MemorySpace.VMEM),
        ],
        out_specs=[
            pl.BlockSpec(memory_space=pltpu.MemorySpace.VMEM),
            pl.BlockSpec(memory_space=pltpu.SMEM),
        ],
        out_shape=[
            jax.ShapeDtypeStruct((8, 128), jnp.float32),
            jax.ShapeDtypeStruct((1, 8), jnp.float32),
        ],
    )(sc, cnt2)


def _final_body(sc_ref, car_ref, cnt_ref, ce_ref, out_ref):
    w = pl.program_id(0)

    @pl.when(w == 0)
    def _():
        cnt = jnp.sum(cnt_ref[...], axis=0, keepdims=True)
        k_rem = car_ref[0, 1]
        cge = _suffix_sum(cnt)
        j = lax.broadcasted_iota(jnp.int32, (1, NB), 1).astype(jnp.float32)
        b3 = jnp.max(jnp.where(cge >= k_rem, j, -1.0))
        cnt_gt3 = jnp.sum(jnp.where(j > b3, cnt, 0.0))
        out_ref[0, 7] = car_ref[0, 2] * float(NB) + b3
        out_ref[0, 8] = k_rem - cnt_gt3
        out_ref[0, 9] = jnp.sum(jnp.where(j == b3, cnt, 0.0))

    kthr = out_ref[0, 7].astype(jnp.int32)

    v = ce_ref[0]
    valid = v >= 0.0
    key = lax.shift_right_logical(lax.bitcast_convert_type(v, jnp.int32), 7)
    zero = jnp.zeros_like(v)
    s_gt = jnp.sum(jnp.where(valid & (key > kthr), v, zero))
    s_eq = jnp.sum(jnp.where(valid & (key == kthr), v, zero))

    @pl.when(w == 0)
    def _():
        out_ref[0, 5] = s_gt
        out_ref[0, 6] = s_eq

    @pl.when(w != 0)
    def _():
        out_ref[0, 5] = out_ref[0, 5] + s_gt
        out_ref[0, 6] = out_ref[0, 6] + s_eq

    @pl.when(w == NW - 1)
    def _():
        k = car_ref[0, 0]
        k3 = out_ref[0, 8]
        cnt_eq3 = out_ref[0, 9]
        mean3 = out_ref[0, 6] / jnp.maximum(cnt_eq3, 1.0)
        topk = out_ref[0, 5] + k3 * mean3

        n_pos = sc_ref[0, 0]
        loss_pos = sc_ref[0, 1]
        avail = sc_ref[0, 2]
        sum_neg_tot = sc_ref[0, 3]
        loss_neg = jnp.where(k >= avail, sum_neg_tot,
                             jnp.where(k <= 0.0, 0.0, topk))
        loss_tr = (loss_pos + loss_neg) / (n_pos + k)

        ttm_cnt = sc_ref[0, 4]
        sum_tcl = sc_ref[0, 5]
        m_cnt = sc_ref[0, 6]
        tct_cnt = sc_ref[0, 7]
        loss_tcl = jnp.where(ttm_cnt > 0,
                             sum_tcl / jnp.maximum(ttm_cnt, 1.0), 0.0)
        gd = jnp.maximum(m_cnt, 1.0)
        out_ref[0, 0] = loss_tr
        out_ref[0, 1] = loss_tcl
        out_ref[0, 2] = jnp.where(tct_cnt > 0, sc_ref[0, 8] / gd, 0.0)
        out_ref[0, 3] = jnp.where(tct_cnt > 0, sc_ref[0, 9] / gd, 0.0)
        out_ref[0, 4] = jnp.where(tct_cnt > 0, sc_ref[0, 10] / gd, 0.0)


def _run_final(sc, carry, cnt3, ce_neg):
    return pl.pallas_call(
        _final_body,
        grid=(NW,),
        in_specs=[
            pl.BlockSpec((1, 16), lambda w: (0, 0), memory_space=pltpu.SMEM),
            pl.BlockSpec((1, 8), lambda w: (0, 0), memory_space=pltpu.SMEM),
            pl.BlockSpec((NW, NB), lambda w: (0, 0),
                         memory_space=pltpu.MemorySpace.VMEM),
            pl.BlockSpec((1, _WR, _W), lambda w: (w, 0, 0)),
        ],
        out_specs=pl.BlockSpec((1, 16), lambda w: (0, 0),
                               memory_space=pltpu.SMEM),
        out_shape=jax.ShapeDtypeStruct((1, 16), jnp.float32),
    )(sc, carry, cnt3, ce_neg)


def kernel(input, tr_mask, tcl_mask, sin_map, cos_map, radii_map, train_mask):
    ce_neg, key, sc = _run_pass1(input, tr_mask, tcl_mask, train_mask,
                                 sin_map, cos_map, radii_map)
    cnt2 = _get_sc_hist(2)(key)
    bnd3, carry = _run_select(sc, cnt2)
    cnt3 = _get_sc_hist(3)(key, bnd3)
    out = _run_final(sc, carry, cnt3, ce_neg)
    return (out[0, 0], out[0, 1], out[0, 2], out[0, 3], out[0, 4])


# speedup vs baseline: 24.0700x; 1.0420x over previous
import functools

import jax
import jax.numpy as jnp
from jax import lax
from jax.experimental import pallas as pl
from jax.experimental.pallas import tpu as pltpu
from jax.experimental.pallas import tpu_sc as plsc

N = 8 * 512 * 512
NB = 4096
NW = 32
_H, _W = 512, 512
_RB = 256
_GB, _GR = 8, _H // _RB
_WPB = _RB // 128
_WR = 128
_TRASH = 4095 << 12
_TBIN = 4080


def _softplus(d):
    return jnp.maximum(d, 0.0) + jnp.log1p(jnp.exp(-jnp.abs(d)))


def _smooth_l1(x, y):
    d = jnp.abs(x - y)
    return jnp.where(d < 1.0, 0.5 * d * d, d - 0.5)


def _pass1_body(x_ref, tr_ref, tcl_ref, trn_ref, sin_ref, cos_ref, rad_ref,
                ce_ref, key_ref, sc_ref):
    b = pl.program_id(0)
    r = pl.program_id(1)

    l0 = x_ref[0, 0]
    l1 = x_ref[0, 1]
    t0 = x_ref[0, 2]
    t1 = x_ref[0, 3]
    sp = x_ref[0, 4]
    cp = x_ref[0, 5]
    rp = x_ref[0, 6]
    tr = tr_ref[0]
    tcl = tcl_ref[0]
    trn = trn_ref[0]
    sinm = sin_ref[0]
    cosm = cos_ref[0]
    radm = rad_ref[0]

    s = l1 - l0
    ce = _softplus(jnp.where(tr == 1, -s, s))
    posm = (tr * trn) != 0
    negm = ((1 - tr) * trn) != 0
    zero = jnp.zeros_like(ce)
    n_pos = jnp.sum(jnp.where(posm, 1.0, 0.0))
    loss_pos = jnp.sum(jnp.where(posm, ce, zero))
    n_negav = jnp.sum(jnp.where(negm, 1.0, 0.0))
    sum_neg = jnp.sum(jnp.where(negm, ce, zero))
    ce_ref[...] = jnp.reshape(jnp.where(negm, ce, jnp.full_like(ce, -1.0)),
                              (_WPB, _WR, _W))
    kbits = lax.shift_right_logical(lax.bitcast_convert_type(ce, jnp.int32), 7)
    key_ref[...] = jnp.reshape(jnp.where(negm, kbits, jnp.full_like(kbits, _TRASH)),
                               (_WPB, _WR, _W))

    st = t1 - t0
    ce_t = _softplus(jnp.where(tcl == 1, -st, st))
    ttmm = (trn * tr) != 0
    ttm_cnt = jnp.sum(jnp.where(ttmm, 1.0, 0.0))
    sum_tcl = jnp.sum(jnp.where(ttmm, ce_t, zero))

    sel = tcl != 0
    m_cnt = jnp.sum(jnp.where(sel, 1.0, 0.0))
    tct_cnt = jnp.sum(jnp.where((trn * tcl) != 0, 1.0, 0.0))
    scale = jnp.sqrt(1.0 / (sp * sp + cp * cp))
    s_rad = jnp.sum(jnp.where(sel, _smooth_l1(rp / radm, jnp.ones_like(rp)), zero))
    s_sin = jnp.sum(jnp.where(sel, _smooth_l1(sp * scale, sinm), zero))
    s_cos = jnp.sum(jnp.where(sel, _smooth_l1(cp * scale, cosm), zero))

    vals = [n_pos, loss_pos, n_negav, sum_neg, ttm_cnt, sum_tcl,
            m_cnt, tct_cnt, s_rad, s_sin, s_cos]
    first = (b == 0) & (r == 0)

    @pl.when(first)
    def _():
        for i, v in enumerate(vals):
            sc_ref[0, i] = v

    @pl.when(jnp.logical_not(first))
    def _():
        for i, v in enumerate(vals):
            sc_ref[0, i] = sc_ref[0, i] + v


def _run_pass1(x, tr, tcl, trn, sinm, cosm, radm):
    m_spec = pl.BlockSpec((1, _RB, _W), lambda b, r: (b, r, 0))
    w_spec = pl.BlockSpec((_WPB, _WR, _W), lambda b, r: (b * _GR + r, 0, 0))
    return pl.pallas_call(
        _pass1_body,
        grid=(_GB, _GR),
        in_specs=[
            pl.BlockSpec((1, 7, _RB, _W), lambda b, r: (b, 0, r, 0)),
            m_spec, m_spec, m_spec, m_spec, m_spec, m_spec,
        ],
        out_specs=[
            w_spec,
            w_spec,
            pl.BlockSpec((1, 16), lambda b, r: (0, 0),
                         memory_space=pltpu.SMEM),
        ],
        out_shape=[
            jax.ShapeDtypeStruct((NW, _WR, _W), jnp.float32),
            jax.ShapeDtypeStruct((NW, _WR, _W), jnp.int32),
            jax.ShapeDtypeStruct((1, 16), jnp.float32),
        ],
    )(x, tr, tcl, trn, sinm, cosm, radm)


def _zero_hist(cnt_v):
    zeros = jnp.zeros((16,), jnp.float32)

    def zbody(i, c):
        for u in range(8):
            cnt_v[pl.ds(i * 128 + u * 16, 16)] = zeros
        return c

    lax.fori_loop(0, NB // 128, zbody, 0)


_CH = 8
_RC = _WR // _CH


def _dbuf_scan(key_hbm, wid, buf, sems, process):
    pend = pltpu.async_copy(
        key_hbm.at[wid, pl.ds(0, _RC)], buf.at[0], sems[0])
    for c in range(_CH):
        nxt = None
        if c + 1 < _CH:
            nxt = pltpu.async_copy(
                key_hbm.at[wid, pl.ds((c + 1) * _RC, _RC)],
                buf.at[(c + 1) % 2], sems[(c + 1) % 2])
        pend.wait()
        process(c % 2)
        pend = nxt


def _sc_hist2_body(key_hbm, cnt_hbm, buf, cnt_v, sem0, sem1):
    wid = lax.axis_index("s") * 2 + lax.axis_index("c")
    _zero_hist(cnt_v)
    ones = jnp.ones((16,), jnp.float32)
    trash = jnp.full((16,), _TRASH, jnp.int32)

    def process(par):
        def row(i, c):
            for u in range(_W // 16):
                k = buf[par, i, pl.ds(u * 16, 16)]
                idx = lax.shift_right_logical(k, 12)
                plsc.addupdate_scatter(cnt_v, [idx], ones, mask=k < trash)
            return c
        lax.fori_loop(0, _RC, row, 0)

    _dbuf_scan(key_hbm, wid, buf, (sem0, sem1), process)
    pltpu.sync_copy(cnt_v, cnt_hbm.at[wid])


def _sc_hist3_body(key_hbm, bnd_hbm, cnt_hbm, buf, cnt_v, bnd_v, sem0, sem1):
    wid = lax.axis_index("s") * 2 + lax.axis_index("c")
    pltpu.sync_copy(bnd_hbm, bnd_v)
    bsel = bnd_v[0, pl.ds(0, 16)].astype(jnp.int32)
    _zero_hist(cnt_v)
    ones = jnp.ones((16,), jnp.float32)
    low = jnp.full((16,), NB - 1, jnp.int32)

    def process(par):
        def row(i, c):
            for u in range(_W // 16):
                k = buf[par, i, pl.ds(u * 16, 16)]
                coarse = lax.shift_right_logical(k, 12)
                idx = jnp.bitwise_and(k, low)
                plsc.addupdate_scatter(cnt_v, [idx], ones,
                                       mask=coarse == bsel)
            return c
        lax.fori_loop(0, _RC, row, 0)

    _dbuf_scan(key_hbm, wid, buf, (sem0, sem1), process)
    pltpu.sync_copy(cnt_v, cnt_hbm.at[wid])


@functools.cache
def _get_sc_hist(level):
    if level == 2:
        body = _sc_hist2_body
        scratch = [
            pltpu.VMEM((2, _RC, _W), jnp.int32),
            pltpu.VMEM((NB,), jnp.float32),
            pltpu.SemaphoreType.DMA,
            pltpu.SemaphoreType.DMA,
        ]
    else:
        body = _sc_hist3_body
        scratch = [
            pltpu.VMEM((2, _RC, _W), jnp.int32),
            pltpu.VMEM((NB,), jnp.float32),
            pltpu.VMEM((8, 128), jnp.float32),
            pltpu.SemaphoreType.DMA,
            pltpu.SemaphoreType.DMA,
        ]
    return pl.kernel(
        body,
        out_type=jax.ShapeDtypeStruct((NW, NB), jnp.float32),
        mesh=plsc.VectorSubcoreMesh(core_axis_name="c", subcore_axis_name="s"),
        compiler_params=pltpu.CompilerParams(needs_layout_passes=False),
        scratch_types=scratch,
    )


def _suffix_sum(x):
    s = x
    off = 1
    while off < NB:
        s = s + jnp.concatenate(
            [s[:, off:], jnp.zeros((1, off), s.dtype)], axis=1)
        off *= 2
    return s


def _select_body(sc_ref, cnt_ref, bnd_ref, car_ref):
    cnt = jnp.sum(cnt_ref[...], axis=0, keepdims=True)
    j = lax.broadcasted_iota(jnp.int32, (1, NB), 1).astype(jnp.float32)
    cnt = jnp.where(j < float(_TBIN), cnt, 0.0)
    n_pos = sc_ref[0, 0]
    avail = sc_ref[0, 2]
    k = jnp.where(n_pos > 0, jnp.minimum(avail, 3.0 * n_pos), 100.0)
    cge = _suffix_sum(cnt)
    b = jnp.max(jnp.where(cge >= k, j, -1.0))
    cnt_gt = jnp.sum(jnp.where(j > b, cnt, 0.0))
    k_rem = k - cnt_gt
    rows = lax.broadcasted_iota(jnp.int32, (8, 128), 0)
    bnd_ref[...] = jnp.where(rows == 0, b, 0.0)
    car_ref[0, 0] = k
    car_ref[0, 1] = k_rem
    car_ref[0, 2] = b
    for i in range(3, 8):
        car_ref[0, i] = 0.0


def _run_select(sc, cnt2):
    return pl.pallas_call(
        _select_body,
        in_specs=[
            pl.BlockSpec(memory_space=pltpu.SMEM),
            pl.BlockSpec(memory_space=pltpu.MemorySpace.VMEM),
        ],
        out_specs=[
            pl.BlockSpec(memory_space=pltpu.MemorySpace.VMEM),
            pl.BlockSpec(memory_space=pltpu.SMEM),
        ],
        out_shape=[
            jax.ShapeDtypeStruct((8, 128), jnp.float32),
            jax.ShapeDtypeStruct((1, 8), jnp.float32),
        ],
    )(sc, cnt2)


def _final_body(sc_ref, car_ref, cnt_ref, ce_ref, out_ref):
    w = pl.program_id(0)

    @pl.when(w == 0)
    def _():
        cnt = jnp.sum(cnt_ref[...], axis=0, keepdims=True)
        k_rem = car_ref[0, 1]
        cge = _suffix_sum(cnt)
        j = lax.broadcasted_iota(jnp.int32, (1, NB), 1).astype(jnp.float32)
        b3 = jnp.max(jnp.where(cge >= k_rem, j, -1.0))
        cnt_gt3 = jnp.sum(jnp.where(j > b3, cnt, 0.0))
        out_ref[0, 7] = car_ref[0, 2] * float(NB) + b3
        out_ref[0, 8] = k_rem - cnt_gt3
        out_ref[0, 9] = jnp.sum(jnp.where(j == b3, cnt, 0.0))

    kthr = out_ref[0, 7].astype(jnp.int32)

    v = ce_ref[0]
    valid = v >= 0.0
    key = lax.shift_right_logical(lax.bitcast_convert_type(v, jnp.int32), 7)
    zero = jnp.zeros_like(v)
    s_gt = jnp.sum(jnp.where(valid & (key > kthr), v, zero))
    s_eq = jnp.sum(jnp.where(valid & (key == kthr), v, zero))

    @pl.when(w == 0)
    def _():
        out_ref[0, 5] = s_gt
        out_ref[0, 6] = s_eq

    @pl.when(w != 0)
    def _():
        out_ref[0, 5] = out_ref[0, 5] + s_gt
        out_ref[0, 6] = out_ref[0, 6] + s_eq

    @pl.when(w == NW - 1)
    def _():
        k = car_ref[0, 0]
        k3 = out_ref[0, 8]
        cnt_eq3 = out_ref[0, 9]
        mean3 = out_ref[0, 6] / jnp.maximum(cnt_eq3, 1.0)
        topk = out_ref[0, 5] + k3 * mean3

        n_pos = sc_ref[0, 0]
        loss_pos = sc_ref[0, 1]
        avail = sc_ref[0, 2]
        sum_neg_tot = sc_ref[0, 3]
        loss_neg = jnp.where(k >= avail, sum_neg_tot,
                             jnp.where(k <= 0.0, 0.0, topk))
        loss_tr = (loss_pos + loss_neg) / (n_pos + k)

        ttm_cnt = sc_ref[0, 4]
        sum_tcl = sc_ref[0, 5]
        m_cnt = sc_ref[0, 6]
        tct_cnt = sc_ref[0, 7]
        loss_tcl = jnp.where(ttm_cnt > 0,
                             sum_tcl / jnp.maximum(ttm_cnt, 1.0), 0.0)
        gd = jnp.maximum(m_cnt, 1.0)
        out_ref[0, 0] = loss_tr
        out_ref[0, 1] = loss_tcl
        out_ref[0, 2] = jnp.where(tct_cnt > 0, sc_ref[0, 8] / gd, 0.0)
        out_ref[0, 3] = jnp.where(tct_cnt > 0, sc_ref[0, 9] / gd, 0.0)
        out_ref[0, 4] = jnp.where(tct_cnt > 0, sc_ref[0, 10] / gd, 0.0)


def _run_final(sc, carry, cnt3, ce_neg):
    return pl.pallas_call(
        _final_body,
        grid=(NW,),
        in_specs=[
            pl.BlockSpec((1, 16), lambda w: (0, 0), memory_space=pltpu.SMEM),
            pl.BlockSpec((1, 8), lambda w: (0, 0), memory_space=pltpu.SMEM),
            pl.BlockSpec((NW, NB), lambda w: (0, 0),
                         memory_space=pltpu.MemorySpace.VMEM),
            pl.BlockSpec((1, _WR, _W), lambda w: (w, 0, 0)),
        ],
        out_specs=pl.BlockSpec((1, 16), lambda w: (0, 0),
                               memory_space=pltpu.SMEM),
        out_shape=jax.ShapeDtypeStruct((1, 16), jnp.float32),
    )(sc, carry, cnt3, ce_neg)


def kernel(input, tr_mask, tcl_mask, sin_map, cos_map, radii_map, train_mask):
    ce_neg, key, sc = _run_pass1(input, tr_mask, tcl_mask, train_mask,
                                 sin_map, cos_map, radii_map)
    cnt2 = _get_sc_hist(2)(key)
    bnd3, carry = _run_select(sc, cnt2)
    cnt3 = _get_sc_hist(3)(key, bnd3)
    out = _run_final(sc, carry, cnt3, ce_neg)
    return (out[0, 0], out[0, 1], out[0, 2], out[0, 3], out[0, 4])


# speedup vs baseline: 24.4509x; 1.0158x over previous
import functools

import jax
import jax.numpy as jnp
from jax import lax
from jax.experimental import pallas as pl
from jax.experimental.pallas import tpu as pltpu
from jax.experimental.pallas import tpu_sc as plsc

N = 8 * 512 * 512
NB = 4096
NW = 32
_H, _W = 512, 512
_RB = 512
_GB, _GR = 8, _H // _RB
_WPB = _RB // 128
_WR = 128
_TRASH = 4095 << 12
_TBIN = 4080


def _softplus(d):
    return jnp.maximum(d, 0.0) + jnp.log1p(jnp.exp(-jnp.abs(d)))


def _smooth_l1(x, y):
    d = jnp.abs(x - y)
    return jnp.where(d < 1.0, 0.5 * d * d, d - 0.5)


def _pass1_body(x_ref, tr_ref, tcl_ref, trn_ref, sin_ref, cos_ref, rad_ref,
                ce_ref, key_ref, sc_ref):
    b = pl.program_id(0)
    r = pl.program_id(1)

    l0 = x_ref[0, 0]
    l1 = x_ref[0, 1]
    t0 = x_ref[0, 2]
    t1 = x_ref[0, 3]
    sp = x_ref[0, 4]
    cp = x_ref[0, 5]
    rp = x_ref[0, 6]
    tr = tr_ref[0]
    tcl = tcl_ref[0]
    trn = trn_ref[0]
    sinm = sin_ref[0]
    cosm = cos_ref[0]
    radm = rad_ref[0]

    s = l1 - l0
    ce = _softplus(jnp.where(tr == 1, -s, s))
    posm = (tr * trn) != 0
    negm = ((1 - tr) * trn) != 0
    zero = jnp.zeros_like(ce)
    n_pos = jnp.sum(jnp.where(posm, 1.0, 0.0))
    loss_pos = jnp.sum(jnp.where(posm, ce, zero))
    n_negav = jnp.sum(jnp.where(negm, 1.0, 0.0))
    sum_neg = jnp.sum(jnp.where(negm, ce, zero))
    ce_ref[...] = jnp.reshape(jnp.where(negm, ce, jnp.full_like(ce, -1.0)),
                              (_WPB, _WR, _W))
    kbits = lax.shift_right_logical(lax.bitcast_convert_type(ce, jnp.int32), 7)
    key_ref[...] = jnp.reshape(jnp.where(negm, kbits, jnp.full_like(kbits, _TRASH)),
                               (_WPB, _WR, _W))

    st = t1 - t0
    ce_t = _softplus(jnp.where(tcl == 1, -st, st))
    ttmm = (trn * tr) != 0
    ttm_cnt = jnp.sum(jnp.where(ttmm, 1.0, 0.0))
    sum_tcl = jnp.sum(jnp.where(ttmm, ce_t, zero))

    sel = tcl != 0
    m_cnt = jnp.sum(jnp.where(sel, 1.0, 0.0))
    tct_cnt = jnp.sum(jnp.where((trn * tcl) != 0, 1.0, 0.0))
    scale = jnp.sqrt(1.0 / (sp * sp + cp * cp))
    s_rad = jnp.sum(jnp.where(sel, _smooth_l1(rp / radm, jnp.ones_like(rp)), zero))
    s_sin = jnp.sum(jnp.where(sel, _smooth_l1(sp * scale, sinm), zero))
    s_cos = jnp.sum(jnp.where(sel, _smooth_l1(cp * scale, cosm), zero))

    vals = [n_pos, loss_pos, n_negav, sum_neg, ttm_cnt, sum_tcl,
            m_cnt, tct_cnt, s_rad, s_sin, s_cos]
    first = (b == 0) & (r == 0)

    @pl.when(first)
    def _():
        for i, v in enumerate(vals):
            sc_ref[0, i] = v

    @pl.when(jnp.logical_not(first))
    def _():
        for i, v in enumerate(vals):
            sc_ref[0, i] = sc_ref[0, i] + v


def _run_pass1(x, tr, tcl, trn, sinm, cosm, radm):
    m_spec = pl.BlockSpec((1, _RB, _W), lambda b, r: (b, r, 0))
    w_spec = pl.BlockSpec((_WPB, _WR, _W), lambda b, r: (b * _GR + r, 0, 0))
    return pl.pallas_call(
        _pass1_body,
        grid=(_GB, _GR),
        in_specs=[
            pl.BlockSpec((1, 7, _RB, _W), lambda b, r: (b, 0, r, 0)),
            m_spec, m_spec, m_spec, m_spec, m_spec, m_spec,
        ],
        out_specs=[
            w_spec,
            w_spec,
            pl.BlockSpec((1, 16), lambda b, r: (0, 0),
                         memory_space=pltpu.SMEM),
        ],
        out_shape=[
            jax.ShapeDtypeStruct((NW, _WR, _W), jnp.float32),
            jax.ShapeDtypeStruct((NW, _WR, _W), jnp.int32),
            jax.ShapeDtypeStruct((1, 16), jnp.float32),
        ],
    )(x, tr, tcl, trn, sinm, cosm, radm)


def _zero_hist(cnt_v):
    zeros = jnp.zeros((16,), jnp.float32)

    def zbody(i, c):
        for u in range(8):
            cnt_v[pl.ds(i * 128 + u * 16, 16)] = zeros
        return c

    lax.fori_loop(0, NB // 128, zbody, 0)


_CH = 8
_RC = _WR // _CH


def _dbuf_scan(key_hbm, wid, buf, sems, process):
    pend = pltpu.async_copy(
        key_hbm.at[wid, pl.ds(0, _RC)], buf.at[0], sems[0])
    for c in range(_CH):
        nxt = None
        if c + 1 < _CH:
            nxt = pltpu.async_copy(
                key_hbm.at[wid, pl.ds((c + 1) * _RC, _RC)],
                buf.at[(c + 1) % 2], sems[(c + 1) % 2])
        pend.wait()
        process(c % 2)
        pend = nxt


def _sc_hist2_body(key_hbm, cnt_hbm, buf, cnt_v, sem0, sem1):
    wid = lax.axis_index("s") * 2 + lax.axis_index("c")
    _zero_hist(cnt_v)
    ones = jnp.ones((16,), jnp.float32)
    trash = jnp.full((16,), _TRASH, jnp.int32)

    def process(par):
        def row(i, c):
            for u in range(_W // 16):
                k = buf[par, i, pl.ds(u * 16, 16)]
                idx = lax.shift_right_logical(k, 12)
                plsc.addupdate_scatter(cnt_v, [idx], ones, mask=k < trash)
            return c
        lax.fori_loop(0, _RC, row, 0)

    _dbuf_scan(key_hbm, wid, buf, (sem0, sem1), process)
    pltpu.sync_copy(cnt_v, cnt_hbm.at[wid])


def _sc_hist3_body(key_hbm, bnd_hbm, cnt_hbm, buf, cnt_v, bnd_v, sem0, sem1):
    wid = lax.axis_index("s") * 2 + lax.axis_index("c")
    pltpu.sync_copy(bnd_hbm, bnd_v)
    bsel = bnd_v[0, pl.ds(0, 16)].astype(jnp.int32)
    _zero_hist(cnt_v)
    ones = jnp.ones((16,), jnp.float32)
    low = jnp.full((16,), NB - 1, jnp.int32)

    def process(par):
        def row(i, c):
            for u in range(_W // 16):
                k = buf[par, i, pl.ds(u * 16, 16)]
                coarse = lax.shift_right_logical(k, 12)
                idx = jnp.bitwise_and(k, low)
                plsc.addupdate_scatter(cnt_v, [idx], ones,
                                       mask=coarse == bsel)
            return c
        lax.fori_loop(0, _RC, row, 0)

    _dbuf_scan(key_hbm, wid, buf, (sem0, sem1), process)
    pltpu.sync_copy(cnt_v, cnt_hbm.at[wid])


@functools.cache
def _get_sc_hist(level):
    if level == 2:
        body = _sc_hist2_body
        scratch = [
            pltpu.VMEM((2, _RC, _W), jnp.int32),
            pltpu.VMEM((NB,), jnp.float32),
            pltpu.SemaphoreType.DMA,
            pltpu.SemaphoreType.DMA,
        ]
    else:
        body = _sc_hist3_body
        scratch = [
            pltpu.VMEM((2, _RC, _W), jnp.int32),
            pltpu.VMEM((NB,), jnp.float32),
            pltpu.VMEM((8, 128), jnp.float32),
            pltpu.SemaphoreType.DMA,
            pltpu.SemaphoreType.DMA,
        ]
    return pl.kernel(
        body,
        out_type=jax.ShapeDtypeStruct((NW, NB), jnp.float32),
        mesh=plsc.VectorSubcoreMesh(core_axis_name="c", subcore_axis_name="s"),
        compiler_params=pltpu.CompilerParams(needs_layout_passes=False),
        scratch_types=scratch,
    )


def _suffix_sum(x):
    s = x
    off = 1
    while off < NB:
        s = s + jnp.concatenate(
            [s[:, off:], jnp.zeros((1, off), s.dtype)], axis=1)
        off *= 2
    return s


def _select_body(sc_ref, cnt_ref, bnd_ref, car_ref):
    cnt = jnp.sum(cnt_ref[...], axis=0, keepdims=True)
    j = lax.broadcasted_iota(jnp.int32, (1, NB), 1).astype(jnp.float32)
    cnt = jnp.where(j < float(_TBIN), cnt, 0.0)
    n_pos = sc_ref[0, 0]
    avail = sc_ref[0, 2]
    k = jnp.where(n_pos > 0, jnp.minimum(avail, 3.0 * n_pos), 100.0)
    cge = _suffix_sum(cnt)
    b = jnp.max(jnp.where(cge >= k, j, -1.0))
    cnt_gt = jnp.sum(jnp.where(j > b, cnt, 0.0))
    k_rem = k - cnt_gt
    rows = lax.broadcasted_iota(jnp.int32, (8, 128), 0)
    bnd_ref[...] = jnp.where(rows == 0, b, 0.0)
    car_ref[0, 0] = k
    car_ref[0, 1] = k_rem
    car_ref[0, 2] = b
    for i in range(3, 8):
        car_ref[0, i] = 0.0


def _run_select(sc, cnt2):
    return pl.pallas_call(
        _select_body,
        in_specs=[
            pl.BlockSpec(memory_space=pltpu.SMEM),
            pl.BlockSpec(memory_space=pltpu.MemorySpace.VMEM),
        ],
        out_specs=[
            pl.BlockSpec(memory_space=pltpu.MemorySpace.VMEM),
            pl.BlockSpec(memory_space=pltpu.SMEM),
        ],
        out_shape=[
            jax.ShapeDtypeStruct((8, 128), jnp.float32),
            jax.ShapeDtypeStruct((1, 8), jnp.float32),
        ],
    )(sc, cnt2)


def _final_body(sc_ref, car_ref, cnt_ref, ce_ref, out_ref):
    w = pl.program_id(0)

    @pl.when(w == 0)
    def _():
        cnt = jnp.sum(cnt_ref[...], axis=0, keepdims=True)
        k_rem = car_ref[0, 1]
        cge = _suffix_sum(cnt)
        j = lax.broadcasted_iota(jnp.int32, (1, NB), 1).astype(jnp.float32)
        b3 = jnp.max(jnp.where(cge >= k_rem, j, -1.0))
        cnt_gt3 = jnp.sum(jnp.where(j > b3, cnt, 0.0))
        out_ref[0, 7] = car_ref[0, 2] * float(NB) + b3
        out_ref[0, 8] = k_rem - cnt_gt3
        out_ref[0, 9] = jnp.sum(jnp.where(j == b3, cnt, 0.0))

    kthr = out_ref[0, 7].astype(jnp.int32)

    v = ce_ref[0]
    valid = v >= 0.0
    key = lax.shift_right_logical(lax.bitcast_convert_type(v, jnp.int32), 7)
    zero = jnp.zeros_like(v)
    s_gt = jnp.sum(jnp.where(valid & (key > kthr), v, zero))
    s_eq = jnp.sum(jnp.where(valid & (key == kthr), v, zero))

    @pl.when(w == 0)
    def _():
        out_ref[0, 5] = s_gt
        out_ref[0, 6] = s_eq

    @pl.when(w != 0)
    def _():
        out_ref[0, 5] = out_ref[0, 5] + s_gt
        out_ref[0, 6] = out_ref[0, 6] + s_eq

    @pl.when(w == NW - 1)
    def _():
        k = car_ref[0, 0]
        k3 = out_ref[0, 8]
        cnt_eq3 = out_ref[0, 9]
        mean3 = out_ref[0, 6] / jnp.maximum(cnt_eq3, 1.0)
        topk = out_ref[0, 5] + k3 * mean3

        n_pos = sc_ref[0, 0]
        loss_pos = sc_ref[0, 1]
        avail = sc_ref[0, 2]
        sum_neg_tot = sc_ref[0, 3]
        loss_neg = jnp.where(k >= avail, sum_neg_tot,
                             jnp.where(k <= 0.0, 0.0, topk))
        loss_tr = (loss_pos + loss_neg) / (n_pos + k)

        ttm_cnt = sc_ref[0, 4]
        sum_tcl = sc_ref[0, 5]
        m_cnt = sc_ref[0, 6]
        tct_cnt = sc_ref[0, 7]
        loss_tcl = jnp.where(ttm_cnt > 0,
                             sum_tcl / jnp.maximum(ttm_cnt, 1.0), 0.0)
        gd = jnp.maximum(m_cnt, 1.0)
        out_ref[0, 0] = loss_tr
        out_ref[0, 1] = loss_tcl
        out_ref[0, 2] = jnp.where(tct_cnt > 0, sc_ref[0, 8] / gd, 0.0)
        out_ref[0, 3] = jnp.where(tct_cnt > 0, sc_ref[0, 9] / gd, 0.0)
        out_ref[0, 4] = jnp.where(tct_cnt > 0, sc_ref[0, 10] / gd, 0.0)


def _run_final(sc, carry, cnt3, ce_neg):
    return pl.pallas_call(
        _final_body,
        grid=(NW,),
        in_specs=[
            pl.BlockSpec((1, 16), lambda w: (0, 0), memory_space=pltpu.SMEM),
            pl.BlockSpec((1, 8), lambda w: (0, 0), memory_space=pltpu.SMEM),
            pl.BlockSpec((NW, NB), lambda w: (0, 0),
                         memory_space=pltpu.MemorySpace.VMEM),
            pl.BlockSpec((1, _WR, _W), lambda w: (w, 0, 0)),
        ],
        out_specs=pl.BlockSpec((1, 16), lambda w: (0, 0),
                               memory_space=pltpu.SMEM),
        out_shape=jax.ShapeDtypeStruct((1, 16), jnp.float32),
    )(sc, carry, cnt3, ce_neg)


def kernel(input, tr_mask, tcl_mask, sin_map, cos_map, radii_map, train_mask):
    ce_neg, key, sc = _run_pass1(input, tr_mask, tcl_mask, train_mask,
                                 sin_map, cos_map, radii_map)
    cnt2 = _get_sc_hist(2)(key)
    bnd3, carry = _run_select(sc, cnt2)
    cnt3 = _get_sc_hist(3)(key, bnd3)
    out = _run_final(sc, carry, cnt3, ce_neg)
    return (out[0, 0], out[0, 1], out[0, 2], out[0, 3], out[0, 4])
